# Initial kernel scaffold; baseline (speedup 1.0000x reference)
import jax
import jax.numpy as jnp
from jax.experimental import pallas as pl


def kernel(pos, x, pos_final_state, x_final_state, pos_interpolated_transition_state, species_initial_state, species_final_state, batch, edge_index, Wself, Wmsg, R1, R2, Wout, Zemb):
    raise NotImplementedError("write your pallas kernel here")


# speedup vs baseline: 1.8122x; 1.8122x over previous
import functools

import jax
import jax.numpy as jnp
import numpy as np
from jax import lax
from jax.experimental import pallas as pl
from jax.experimental.pallas import tpu as pltpu
from jax.experimental.pallas import tpu_sc as plsc

N = 10000
E = 320000
D = 128
NB = 16
RN = 64
MAXR = 5.0
NNEI = 32.0
NSPEC = 10

NC = 2
NS = 16
NW = NC * NS
EW = E // NW
CH = 80
NCHUNK = EW // CH
NPAD = 10240
ROWS_PER_TILE = NPAD // NS

_mesh = plsc.VectorSubcoreMesh(
    core_axis_name="c", subcore_axis_name="s", num_cores=NC, num_subcores=NS)


def _worker_id():
  return lax.axis_index("c") * NS + lax.axis_index("s")


def _make_sc_gather(nt, dt):
  @functools.partial(
      pl.kernel,
      out_type=jax.ShapeDtypeStruct((E, dt), jnp.float32),
      mesh=_mesh,
      scratch_types=[
          pltpu.VMEM((NCHUNK, CH), jnp.int32),
          pltpu.VMEM((CH, dt), jnp.float32),
          pltpu.VMEM((CH, dt), jnp.float32),
          pltpu.SemaphoreType.DMA,
          pltpu.SemaphoreType.DMA,
      ],
  )
  def gather_kernel(table_hbm, idx_hbm, out_hbm, idx_v, buf0, buf1, sem0, sem1):
    wid = _worker_id()
    pltpu.sync_copy(idx_hbm.at[wid], idx_v)
    ebase = wid * EW
    bufs = (buf0, buf1)
    sems = (sem0, sem1)
    pltpu.async_copy(table_hbm.at[idx_v.at[0]], buf0, sem0)

    def body(j, _):
      for p in range(2):
        jj = 2 * j + p
        nxt = bufs[1 - p]
        nsem = sems[1 - p]
        pltpu.async_copy(table_hbm.at[idx_v.at[jj + 1]], nxt, nsem)
        pltpu.make_async_copy(table_hbm.at[idx_v.at[jj]], bufs[p], sems[p]).wait()
        pltpu.sync_copy(bufs[p], out_hbm.at[pl.ds(ebase + jj * CH, CH)])
      return 0

    lax.fori_loop(0, (NCHUNK - 1) // 2, body, 0)
    pltpu.make_async_copy(
        table_hbm.at[idx_v.at[NCHUNK - 1]], buf0, sem0).wait()
    pltpu.sync_copy(buf0, out_hbm.at[pl.ds(ebase + (NCHUNK - 1) * CH, CH)])

  return gather_kernel


def _make_sc_scatter():
  @functools.partial(
      pl.kernel,
      out_type=jax.ShapeDtypeStruct((NC, NPAD, D), jnp.float32),
      mesh=_mesh,
      scratch_types=[
          pltpu.VMEM((NCHUNK, CH), jnp.int32),
          pltpu.VMEM((CH, D), jnp.float32),
          pltpu.VMEM_SHARED((NPAD, D), jnp.float32),
          pltpu.SemaphoreType.DMA,
      ],
  )
  def scatter_kernel(msg_hbm, idx_hbm, zeros_hbm, out_hbm, idx_v, buf,
                     acc_sh, sem):
    c = lax.axis_index("c")
    s = lax.axis_index("s")
    wid = c * NS + s
    rbase = s * ROWS_PER_TILE
    pltpu.sync_copy(zeros_hbm.at[pl.ds(rbase, ROWS_PER_TILE)],
                    acc_sh.at[pl.ds(rbase, ROWS_PER_TILE)])
    plsc.subcore_barrier()
    pltpu.sync_copy(idx_hbm.at[wid], idx_v)
    ebase = wid * EW

    def body(j, _):
      pltpu.sync_copy(msg_hbm.at[pl.ds(ebase + j * CH, CH)], buf)
      pltpu.sync_copy(buf, acc_sh.at[idx_v.at[j]], add=True)
      return 0

    lax.fori_loop(0, NCHUNK, body, 0)
    plsc.subcore_barrier()
    pltpu.sync_copy(acc_sh.at[pl.ds(rbase, ROWS_PER_TILE)],
                    out_hbm.at[c, pl.ds(rbase, ROWS_PER_TILE)])

  return scatter_kernel


BE = 2000
_SIG = MAXR / NB
_INV2SIG2 = np.float32(1.0 / (2.0 * _SIG * _SIG))
_CSTEP = np.float32(MAXR / (NB - 1))


def _make_tc_msg(net):
  def body(ps_ref, pd_ref, hs_ref, r1_ref, r2_ref, out_ref):
    col = lax.broadcasted_iota(jnp.int32, (1, D), 1)
    mask = ((col >= 3 * net) & (col < 3 * net + 3)).astype(jnp.float32)
    centers = lax.broadcasted_iota(jnp.int32, (1, NB), 1).astype(
        jnp.float32) * _CSTEP
    diff = ps_ref[...] - pd_ref[...]
    d2 = jnp.sum(diff * diff * mask, axis=1, keepdims=True) + 1e-12
    r = jnp.sqrt(d2)
    rb = jnp.exp(-((r - centers) ** 2) * _INV2SIG2)
    cut = 0.5 * (jnp.cos(jnp.pi * jnp.clip(r * (1.0 / MAXR), 0.0, 1.0)) + 1.0)
    rb = rb * cut
    u = jnp.dot(rb, r1_ref[...], preferred_element_type=jnp.float32)
    u = u * (1.0 / (1.0 + jnp.exp(-u)))
    w = jnp.dot(u, r2_ref[...], preferred_element_type=jnp.float32)
    out_ref[...] = hs_ref[...] * w

  return pl.pallas_call(
      body,
      grid=(E // BE,),
      in_specs=[
          pl.BlockSpec((BE, D), lambda i: (i, 0)),
          pl.BlockSpec((BE, D), lambda i: (i, 0)),
          pl.BlockSpec((BE, D), lambda i: (i, 0)),
          pl.BlockSpec((NB, RN), lambda i: (0, 0)),
          pl.BlockSpec((RN, D), lambda i: (0, 0)),
      ],
      out_specs=pl.BlockSpec((BE, D), lambda i: (i, 0)),
      out_shape=jax.ShapeDtypeStruct((E, D), jnp.float32),
  )


BN = 1000


def _make_tc_node(use_avg):
  def body(ha_ref, hb_ref, a0_ref, a1_ref, z_ref, ws_ref, wm_ref, ze_ref,
           out_ref):
    if use_avg:
      h = (ha_ref[...] + hb_ref[...]) * 0.5
    else:
      h = ha_ref[...]
    agg = (a0_ref[...] + a1_ref[...]) * np.float32(1.0 / np.sqrt(NNEI))
    z = z_ref[...]
    spec = lax.broadcasted_iota(jnp.int32, (BN, NSPEC), 1)
    oneh = (z == spec).astype(jnp.float32)
    acc = (jnp.dot(h, ws_ref[...], preferred_element_type=jnp.float32)
           + jnp.dot(agg, wm_ref[...], preferred_element_type=jnp.float32)
           + jnp.dot(oneh, ze_ref[...], preferred_element_type=jnp.float32))
    out_ref[...] = acc * (1.0 / (1.0 + jnp.exp(-acc)))

  return pl.pallas_call(
      body,
      grid=(N // BN,),
      in_specs=[
          pl.BlockSpec((BN, D), lambda i: (i, 0)),
          pl.BlockSpec((BN, D), lambda i: (i, 0)),
          pl.BlockSpec((BN, D), lambda i: (i, 0)),
          pl.BlockSpec((BN, D), lambda i: (i, 0)),
          pl.BlockSpec((BN, 1), lambda i: (i, 0)),
          pl.BlockSpec((D, D), lambda i: (0, 0)),
          pl.BlockSpec((D, D), lambda i: (0, 0)),
          pl.BlockSpec((NSPEC, D), lambda i: (0, 0)),
      ],
      out_specs=pl.BlockSpec((BN, D), lambda i: (i, 0)),
      out_shape=jax.ShapeDtypeStruct((N, D), jnp.float32),
  )


def _make_tc_head(take_abs):
  def body(h_ref, w_ref, out_ref):
    acc = jnp.dot(h_ref[...], w_ref[...], preferred_element_type=jnp.float32)
    out_ref[...] = jnp.abs(acc) if take_abs else acc

  return pl.pallas_call(
      body,
      grid=(N // BN,),
      in_specs=[
          pl.BlockSpec((BN, D), lambda i: (i, 0)),
          pl.BlockSpec((D, D), lambda i: (0, 0)),
      ],
      out_specs=pl.BlockSpec((BN, D), lambda i: (i, 0)),
      out_shape=jax.ShapeDtypeStruct((N, D), jnp.float32),
  )


def _make_tc_avg():
  def body(a_ref, b_ref, out_ref):
    out_ref[...] = (a_ref[...] + b_ref[...]) * 0.5

  return pl.pallas_call(
      body,
      grid=(N // BN,),
      in_specs=[
          pl.BlockSpec((BN, D), lambda i: (i, 0)),
          pl.BlockSpec((BN, D), lambda i: (i, 0)),
      ],
      out_specs=pl.BlockSpec((BN, D), lambda i: (i, 0)),
      out_shape=jax.ShapeDtypeStruct((N, D), jnp.float32),
  )


_gather_pos = _make_sc_gather(N, D)
_gather_h = _make_sc_gather(N, D)
_scatter = _make_sc_scatter()
_msg_k = [_make_tc_msg(net) for net in range(3)]
_node_k = _make_tc_node(False)
_node_avg_k = _make_tc_node(True)
_head_k = _make_tc_head(False)
_head_abs_k = _make_tc_head(True)
_avg_k = _make_tc_avg()


def kernel(pos, x, pos_final_state, x_final_state,
           pos_interpolated_transition_state, species_initial_state,
           species_final_state, batch, edge_index, Wself, Wmsg, R1, R2, Wout,
           Zemb):
  postab = jnp.concatenate(
      [pos, pos_final_state, pos_interpolated_transition_state,
       jnp.zeros((N, D - 9), jnp.float32)], axis=1)
  src3 = edge_index[0].astype(jnp.int32).reshape(NW, NCHUNK, CH)
  dst3 = edge_index[1].astype(jnp.int32).reshape(NW, NCHUNK, CH)
  z_init = species_initial_state.astype(jnp.int32).reshape(N, 1)
  z_final = species_final_state.astype(jnp.int32).reshape(N, 1)
  zeros_nd = jnp.zeros((N, D), jnp.float32)
  zeros_pad = jnp.zeros((NPAD, D), jnp.float32)

  possrc = _gather_pos(postab, src3)
  posdst = _gather_pos(postab, dst3)

  def run_net(net, h, z2d, dummy):
    for l in range(2):
      hsrc = _gather_h(h, src3)
      msg = _msg_k[net](possrc, posdst, hsrc, R1[net, l], R2[net, l])
      agg2 = _scatter(msg, dst3, zeros_pad)[:, :N]
      h = _node_k(h, dummy, agg2[0], agg2[1], z2d, Wself[net, l],
                  Wmsg[net, l], Zemb[net])
    return h

  dummy = zeros_nd
  h0 = run_net(0, x, z_init, dummy)
  out_init = _head_k(h0, Wout[0])
  h1 = run_net(1, x_final_state, z_final, dummy)
  out_final = _head_k(h1, Wout[1])
  x_ts = _avg_k(out_init, out_final)
  h2 = run_net(2, x_ts, z_init, dummy)
  return _head_abs_k(h2, Wout[2])


# speedup vs baseline: 2.6932x; 1.4862x over previous
import functools

import jax
import jax.numpy as jnp
import numpy as np
from jax import lax
from jax.experimental import pallas as pl
from jax.experimental.pallas import tpu as pltpu
from jax.experimental.pallas import tpu_sc as plsc

N = 10000
E = 320000
D = 128
NB = 16
RN = 64
MAXR = 5.0
NNEI = 32.0
NSPEC = 10

NC = 2
NS = 16
NW = NC * NS
EW = E // NW
CH = 80
NCHUNK = EW // CH
NPAD = 10240
ROWS_PER_TILE = NPAD // NS

_mesh = plsc.VectorSubcoreMesh(
    core_axis_name="c", subcore_axis_name="s", num_cores=NC, num_subcores=NS)


def _worker_id():
  return lax.axis_index("c") * NS + lax.axis_index("s")


def _make_sc_gather():
  @functools.partial(
      pl.kernel,
      out_type=jax.ShapeDtypeStruct((E, D), jnp.float32),
      mesh=_mesh,
      scratch_types=[
          pltpu.VMEM((NCHUNK, CH), jnp.int32),
          pltpu.VMEM((CH, D), jnp.float32),
          pltpu.VMEM((CH, D), jnp.float32),
          pltpu.SemaphoreType.DMA,
          pltpu.SemaphoreType.DMA,
      ],
  )
  def gather_kernel(table_hbm, idx_hbm, out_hbm, idx_v, buf0, buf1, sem0, sem1):
    wid = _worker_id()
    pltpu.sync_copy(idx_hbm.at[wid], idx_v)
    ebase = wid * EW
    bufs = (buf0, buf1)
    sems = (sem0, sem1)
    pltpu.async_copy(table_hbm.at[idx_v.at[0]], buf0, sem0)

    def body(j, _):
      for p in range(2):
        jj = 2 * j + p
        pltpu.async_copy(table_hbm.at[idx_v.at[jj + 1]], bufs[1 - p],
                         sems[1 - p])
        pltpu.make_async_copy(table_hbm.at[idx_v.at[jj]], bufs[p],
                              sems[p]).wait()
        pltpu.sync_copy(bufs[p], out_hbm.at[pl.ds(ebase + jj * CH, CH)])
      return 0

    lax.fori_loop(0, (NCHUNK - 1) // 2, body, 0)
    pltpu.make_async_copy(
        table_hbm.at[idx_v.at[NCHUNK - 1]], buf0, sem0).wait()
    pltpu.sync_copy(buf0, out_hbm.at[pl.ds(ebase + (NCHUNK - 1) * CH, CH)])

  return gather_kernel


CH2 = 40
NCH2 = EW // CH2
WWIN = 50
NWIN = NCH2 // WWIN


def _make_sc_fused(which):
  wbase0 = which * E

  @functools.partial(
      pl.kernel,
      out_type=jax.ShapeDtypeStruct((NC, NPAD, D), jnp.float32),
      mesh=_mesh,
      scratch_types=[
          pltpu.VMEM((WWIN, CH2), jnp.int32),
          pltpu.VMEM((WWIN, CH2), jnp.int32),
          pltpu.VMEM((CH2, D), jnp.float32),
          pltpu.VMEM((CH2, D), jnp.float32),
          pltpu.VMEM((CH2, D), jnp.float32),
          pltpu.VMEM((CH2, D), jnp.float32),
          pltpu.VMEM_SHARED((NPAD, D), jnp.float32),
          pltpu.SemaphoreType.DMA,
          pltpu.SemaphoreType.DMA,
      ],
  )
  def fused_kernel(w6_hbm, h_hbm, src_hbm, dst_hbm, zeros_hbm, out_hbm,
                   src_v, dst_v, wb0, wb1, hb0, hb1, acc_sh, sem0, sem1):
    c = lax.axis_index("c")
    s = lax.axis_index("s")
    wid = c * NS + s
    rbase = s * ROWS_PER_TILE
    pltpu.sync_copy(zeros_hbm.at[pl.ds(rbase, ROWS_PER_TILE)],
                    acc_sh.at[pl.ds(rbase, ROWS_PER_TILE)])
    plsc.subcore_barrier()
    ebase = wid * EW
    wbase = wbase0 + ebase
    wbufs = (wb0, wb1)
    hbufs = (hb0, hb1)
    sems = (sem0, sem1)

    def win_body(win, _):
      pltpu.sync_copy(src_hbm.at[wid, win], src_v)
      pltpu.sync_copy(dst_hbm.at[wid, win], dst_v)
      wb_e = wbase + win * (WWIN * CH2)

      def issue(j, p):
        pltpu.async_copy(w6_hbm.at[pl.ds(wb_e + j * CH2, CH2)], wbufs[p],
                         sems[p])
        pltpu.async_copy(h_hbm.at[src_v.at[j]], hbufs[p], sems[p])

      def wait(j, p):
        pltpu.make_async_copy(w6_hbm.at[pl.ds(wb_e + j * CH2, CH2)],
                              wbufs[p], sems[p]).wait()
        pltpu.make_async_copy(h_hbm.at[src_v.at[j]], hbufs[p],
                              sems[p]).wait()

      def process(j, p):
        wb = wbufs[p]
        hb = hbufs[p]

        def mrow(i, _):
          for k in range(D // 16):
            sl = pl.ds(k * 16, 16)
            wb[i, sl] = wb[i, sl] * hb[i, sl]
          return 0

        lax.fori_loop(0, CH2, mrow, 0)
        pltpu.sync_copy(wb, acc_sh.at[dst_v.at[j]], add=True)

      issue(0, 0)

      def pair(i, _):
        j0 = 2 * i
        issue(j0 + 1, 1)
        wait(j0, 0)
        process(j0, 0)

        @pl.when(j0 + 2 < WWIN)
        def _():
          issue(j0 + 2, 0)

        wait(j0 + 1, 1)
        process(j0 + 1, 1)
        return 0

      lax.fori_loop(0, WWIN // 2, pair, 0)
      return 0

    lax.fori_loop(0, NWIN, win_body, 0)
    plsc.subcore_barrier()
    pltpu.sync_copy(acc_sh.at[pl.ds(rbase, ROWS_PER_TILE)],
                    out_hbm.at[c, pl.ds(rbase, ROWS_PER_TILE)])

  return fused_kernel


BE = 2000
_SIG = MAXR / NB
_INV2SIG2 = np.float32(1.0 / (2.0 * _SIG * _SIG))
_CSTEP = np.float32(MAXR / (NB - 1))


def _make_tc_wall():
  def body(ps_ref, pd_ref, r1_ref, r2_ref, out_ref):
    col = lax.broadcasted_iota(jnp.int32, (1, D), 1)
    centers = lax.broadcasted_iota(jnp.int32, (1, NB), 1).astype(
        jnp.float32) * _CSTEP
    diff = ps_ref[...] - pd_ref[...]
    dsq = diff * diff
    rbs = []
    for net in range(3):
      mask = ((col >= 3 * net) & (col < 3 * net + 3)).astype(jnp.float32)
      d2 = jnp.sum(dsq * mask, axis=1, keepdims=True) + 1e-12
      r = jnp.sqrt(d2)
      rb = jnp.exp(-((r - centers) ** 2) * _INV2SIG2)
      cut = 0.5 * (jnp.cos(jnp.pi * jnp.clip(r * (1.0 / MAXR), 0.0, 1.0))
                   + 1.0)
      rbs.append(rb * cut)
    for m in range(6):
      u = jnp.dot(rbs[m // 2], r1_ref[m], preferred_element_type=jnp.float32)
      u = u * (1.0 / (1.0 + jnp.exp(-u)))
      out_ref[m] = jnp.dot(u, r2_ref[m], preferred_element_type=jnp.float32)

  return pl.pallas_call(
      body,
      grid=(E // BE,),
      in_specs=[
          pl.BlockSpec((BE, D), lambda i: (i, 0)),
          pl.BlockSpec((BE, D), lambda i: (i, 0)),
          pl.BlockSpec((6, NB, RN), lambda i: (0, 0, 0)),
          pl.BlockSpec((6, RN, D), lambda i: (0, 0, 0)),
      ],
      out_specs=pl.BlockSpec((6, BE, D), lambda i: (0, i, 0)),
      out_shape=jax.ShapeDtypeStruct((6, E, D), jnp.float32),
  )


BN = 1000


def _make_tc_node(use_avg):
  def body(ha_ref, hb_ref, a0_ref, a1_ref, z_ref, ws_ref, wm_ref, ze_ref,
           out_ref):
    if use_avg:
      h = (ha_ref[...] + hb_ref[...]) * 0.5
    else:
      h = ha_ref[...]
    agg = (a0_ref[...] + a1_ref[...]) * np.float32(1.0 / np.sqrt(NNEI))
    z = z_ref[...]
    spec = lax.broadcasted_iota(jnp.int32, (BN, NSPEC), 1)
    oneh = (z == spec).astype(jnp.float32)
    acc = (jnp.dot(h, ws_ref[...], preferred_element_type=jnp.float32)
           + jnp.dot(agg, wm_ref[...], preferred_element_type=jnp.float32)
           + jnp.dot(oneh, ze_ref[...], preferred_element_type=jnp.float32))
    out_ref[...] = acc * (1.0 / (1.0 + jnp.exp(-acc)))

  return pl.pallas_call(
      body,
      grid=(N // BN,),
      in_specs=[
          pl.BlockSpec((BN, D), lambda i: (i, 0)),
          pl.BlockSpec((BN, D), lambda i: (i, 0)),
          pl.BlockSpec((BN, D), lambda i: (i, 0)),
          pl.BlockSpec((BN, D), lambda i: (i, 0)),
          pl.BlockSpec((BN, 1), lambda i: (i, 0)),
          pl.BlockSpec((D, D), lambda i: (0, 0)),
          pl.BlockSpec((D, D), lambda i: (0, 0)),
          pl.BlockSpec((NSPEC, D), lambda i: (0, 0)),
      ],
      out_specs=pl.BlockSpec((BN, D), lambda i: (i, 0)),
      out_shape=jax.ShapeDtypeStruct((N, D), jnp.float32),
  )


def _make_tc_head(take_abs):
  def body(h_ref, w_ref, out_ref):
    acc = jnp.dot(h_ref[...], w_ref[...], preferred_element_type=jnp.float32)
    out_ref[...] = jnp.abs(acc) if take_abs else acc

  return pl.pallas_call(
      body,
      grid=(N // BN,),
      in_specs=[
          pl.BlockSpec((BN, D), lambda i: (i, 0)),
          pl.BlockSpec((D, D), lambda i: (0, 0)),
      ],
      out_specs=pl.BlockSpec((BN, D), lambda i: (i, 0)),
      out_shape=jax.ShapeDtypeStruct((N, D), jnp.float32),
  )


def _make_tc_avg():
  def body(a_ref, b_ref, out_ref):
    out_ref[...] = (a_ref[...] + b_ref[...]) * 0.5

  return pl.pallas_call(
      body,
      grid=(N // BN,),
      in_specs=[
          pl.BlockSpec((BN, D), lambda i: (i, 0)),
          pl.BlockSpec((BN, D), lambda i: (i, 0)),
      ],
      out_specs=pl.BlockSpec((BN, D), lambda i: (i, 0)),
      out_shape=jax.ShapeDtypeStruct((N, D), jnp.float32),
  )


_gather_pos = _make_sc_gather()
_fused_k = [_make_sc_fused(k) for k in range(6)]
_wall_k = _make_tc_wall()
_node_k = _make_tc_node(False)
_node_avg_k = _make_tc_node(True)
_head_k = _make_tc_head(False)
_head_abs_k = _make_tc_head(True)
_avg_k = _make_tc_avg()


def kernel(pos, x, pos_final_state, x_final_state,
           pos_interpolated_transition_state, species_initial_state,
           species_final_state, batch, edge_index, Wself, Wmsg, R1, R2, Wout,
           Zemb):
  postab = jnp.concatenate(
      [pos, pos_final_state, pos_interpolated_transition_state,
       jnp.zeros((N, D - 9), jnp.float32)], axis=1)
  src_i = edge_index[0].astype(jnp.int32)
  dst_i = edge_index[1].astype(jnp.int32)
  src3 = src_i.reshape(NW, NCHUNK, CH)
  dst3 = dst_i.reshape(NW, NCHUNK, CH)
  src4 = src_i.reshape(NW, NWIN, WWIN, CH2)
  dst4 = dst_i.reshape(NW, NWIN, WWIN, CH2)
  z_init = species_initial_state.astype(jnp.int32).reshape(N, 1)
  z_final = species_final_state.astype(jnp.int32).reshape(N, 1)
  zeros_nd = jnp.zeros((N, D), jnp.float32)
  zeros_pad = jnp.zeros((NPAD, D), jnp.float32)

  possrc = _gather_pos(postab, src3)
  posdst = _gather_pos(postab, dst3)
  r1all = R1.reshape(6, NB, RN)
  r2all = R2.reshape(6, RN, D)
  wall = _wall_k(possrc, posdst, r1all, r2all).reshape(6 * E, D)

  def run_net(net, h, z2d):
    for l in range(2):
      agg2 = _fused_k[2 * net + l](wall, h, src4, dst4, zeros_pad)[:, :N]
      h = _node_k(h, zeros_nd, agg2[0], agg2[1], z2d, Wself[net, l],
                  Wmsg[net, l], Zemb[net])
    return h

  h0 = run_net(0, x, z_init)
  out_init = _head_k(h0, Wout[0])
  h1 = run_net(1, x_final_state, z_final)
  out_final = _head_k(h1, Wout[1])
  x_ts = _avg_k(out_init, out_final)
  h2 = run_net(2, x_ts, z_init)
  return _head_abs_k(h2, Wout[2])


# speedup vs baseline: 2.8381x; 1.0538x over previous
import functools

import jax
import jax.numpy as jnp
import numpy as np
from jax import lax
from jax.experimental import pallas as pl
from jax.experimental.pallas import tpu as pltpu
from jax.experimental.pallas import tpu_sc as plsc

N = 10000
E = 320000
D = 128
NB = 16
RN = 64
MAXR = 5.0
NNEI = 32.0
NSPEC = 10

NC = 2
NS = 16
NW = NC * NS
EW = E // NW
CH = 80
NCHUNK = EW // CH
NPAD = 10240
ROWS_PER_TILE = NPAD // NS

_mesh = plsc.VectorSubcoreMesh(
    core_axis_name="c", subcore_axis_name="s", num_cores=NC, num_subcores=NS)


def _worker_id():
  return lax.axis_index("c") * NS + lax.axis_index("s")


def _make_sc_gather2():
  @functools.partial(
      pl.kernel,
      out_type=(jax.ShapeDtypeStruct((E, D), jnp.float32),
                jax.ShapeDtypeStruct((E, D), jnp.float32)),
      mesh=_mesh,
      scratch_types=[
          pltpu.VMEM((NCHUNK, CH), jnp.int32),
          pltpu.VMEM((NCHUNK, CH), jnp.int32),
          pltpu.VMEM((CH, D), jnp.float32),
          pltpu.VMEM((CH, D), jnp.float32),
          pltpu.VMEM((CH, D), jnp.float32),
          pltpu.VMEM((CH, D), jnp.float32),
          pltpu.SemaphoreType.DMA,
          pltpu.SemaphoreType.DMA,
      ],
  )
  def gather_kernel(table_hbm, sidx_hbm, didx_hbm, outs_hbm, outd_hbm,
                    sidx_v, didx_v, sbuf0, sbuf1, dbuf0, dbuf1, sem0, sem1):
    wid = _worker_id()
    pltpu.sync_copy(sidx_hbm.at[wid], sidx_v)
    pltpu.sync_copy(didx_hbm.at[wid], didx_v)
    ebase = wid * EW
    sbufs = (sbuf0, sbuf1)
    dbufs = (dbuf0, dbuf1)
    sems = (sem0, sem1)

    def issue(j, p):
      pltpu.async_copy(table_hbm.at[sidx_v.at[j]], sbufs[p], sems[p])
      pltpu.async_copy(table_hbm.at[didx_v.at[j]], dbufs[p], sems[p])

    def drain(j, p):
      pltpu.make_async_copy(table_hbm.at[sidx_v.at[j]], sbufs[p],
                            sems[p]).wait()
      pltpu.make_async_copy(table_hbm.at[didx_v.at[j]], dbufs[p],
                            sems[p]).wait()
      pltpu.sync_copy(sbufs[p], outs_hbm.at[pl.ds(ebase + j * CH, CH)])
      pltpu.sync_copy(dbufs[p], outd_hbm.at[pl.ds(ebase + j * CH, CH)])

    issue(0, 0)

    def body(j, _):
      for p in range(2):
        jj = 2 * j + p
        issue(jj + 1, 1 - p)
        drain(jj, p)
      return 0

    lax.fori_loop(0, (NCHUNK - 1) // 2, body, 0)
    drain(NCHUNK - 1, 0)

  return gather_kernel


CH2 = 40
NCH2 = EW // CH2
WWIN = 50
NWIN = NCH2 // WWIN


def _make_sc_fused(which):
  wbase0 = which * E

  @functools.partial(
      pl.kernel,
      out_type=jax.ShapeDtypeStruct((NC, NPAD, D), jnp.float32),
      mesh=_mesh,
      scratch_types=[
          pltpu.VMEM((WWIN, CH2), jnp.int32),
          pltpu.VMEM((WWIN, CH2), jnp.int32),
          pltpu.VMEM((CH2, D), jnp.float32),
          pltpu.VMEM((CH2, D), jnp.float32),
          pltpu.VMEM((CH2, D), jnp.float32),
          pltpu.VMEM((CH2, D), jnp.float32),
          pltpu.VMEM((CH2, D), jnp.float32),
          pltpu.VMEM((CH2, D), jnp.float32),
          pltpu.VMEM_SHARED((NPAD, D), jnp.float32),
          pltpu.SemaphoreType.DMA,
          pltpu.SemaphoreType.DMA,
          pltpu.SemaphoreType.DMA,
          pltpu.SemaphoreType.DMA,
      ],
  )
  def fused_kernel(w6_hbm, h_hbm, src_hbm, dst_hbm, zeros_hbm, out_hbm,
                   src_v, dst_v, wb0, wb1, hb0, hb1, mb0, mb1, acc_sh,
                   sem0, sem1, scs0, scs1):
    c = lax.axis_index("c")
    s = lax.axis_index("s")
    wid = c * NS + s
    rbase = s * ROWS_PER_TILE
    pltpu.sync_copy(zeros_hbm.at[pl.ds(rbase, ROWS_PER_TILE)],
                    acc_sh.at[pl.ds(rbase, ROWS_PER_TILE)])
    plsc.subcore_barrier()
    ebase = wid * EW
    wbase = wbase0 + ebase
    wbufs = (wb0, wb1)
    hbufs = (hb0, hb1)
    mbufs = (mb0, mb1)
    sems = (sem0, sem1)
    scsems = (scs0, scs1)

    def win_body(win, _):
      pltpu.sync_copy(src_hbm.at[wid, win], src_v)
      pltpu.sync_copy(dst_hbm.at[wid, win], dst_v)
      wb_e = wbase + win * (WWIN * CH2)

      def issue(j, p):
        pltpu.async_copy(w6_hbm.at[pl.ds(wb_e + j * CH2, CH2)], wbufs[p],
                         sems[p])
        pltpu.async_copy(h_hbm.at[src_v.at[j]], hbufs[p], sems[p])

      def wait(j, p):
        pltpu.make_async_copy(w6_hbm.at[pl.ds(wb_e + j * CH2, CH2)],
                              wbufs[p], sems[p]).wait()
        pltpu.make_async_copy(h_hbm.at[src_v.at[j]], hbufs[p],
                              sems[p]).wait()

      def wait_sc(j, p):
        pltpu.make_async_copy(mbufs[p], acc_sh.at[dst_v.at[j]],
                              scsems[p]).wait()

      def process(j, p):
        wb = wbufs[p]
        hb = hbufs[p]
        mb = mbufs[p]

        def mrow(i, _):
          for k in range(D // 16):
            sl = pl.ds(k * 16, 16)
            mb[i, sl] = wb[i, sl] * hb[i, sl]
          return 0

        lax.fori_loop(0, CH2, mrow, 0)
        pltpu.async_copy(mb, acc_sh.at[dst_v.at[j]], scsems[p], add=True)

      issue(0, 0)

      def pair(i, _):
        j0 = 2 * i
        issue(j0 + 1, 1)
        wait(j0, 0)

        @pl.when(i > 0)
        def _():
          wait_sc(j0 - 2, 0)

        process(j0, 0)

        @pl.when(j0 + 2 < WWIN)
        def _():
          issue(j0 + 2, 0)

        wait(j0 + 1, 1)

        @pl.when(i > 0)
        def _():
          wait_sc(j0 - 1, 1)

        process(j0 + 1, 1)
        return 0

      lax.fori_loop(0, WWIN // 2, pair, 0)
      wait_sc(WWIN - 2, 0)
      wait_sc(WWIN - 1, 1)
      return 0

    lax.fori_loop(0, NWIN, win_body, 0)
    plsc.subcore_barrier()
    pltpu.sync_copy(acc_sh.at[pl.ds(rbase, ROWS_PER_TILE)],
                    out_hbm.at[c, pl.ds(rbase, ROWS_PER_TILE)])

  return fused_kernel


BE = 2000
_SIG = MAXR / NB
_INV2SIG2 = np.float32(1.0 / (2.0 * _SIG * _SIG))
_CSTEP = np.float32(MAXR / (NB - 1))


def _make_tc_wall():
  def body(ps_ref, pd_ref, r1_ref, r2_ref, out_ref):
    col = lax.broadcasted_iota(jnp.int32, (1, D), 1)
    centers = lax.broadcasted_iota(jnp.int32, (1, NB), 1).astype(
        jnp.float32) * _CSTEP
    diff = ps_ref[...] - pd_ref[...]
    dsq = diff * diff
    rbs = []
    for net in range(3):
      mask = ((col >= 3 * net) & (col < 3 * net + 3)).astype(jnp.float32)
      d2 = jnp.sum(dsq * mask, axis=1, keepdims=True) + 1e-12
      r = jnp.sqrt(d2)
      rb = jnp.exp(-((r - centers) ** 2) * _INV2SIG2)
      cut = 0.5 * (jnp.cos(jnp.pi * jnp.clip(r * (1.0 / MAXR), 0.0, 1.0))
                   + 1.0)
      rbs.append(rb * cut)
    for m in range(6):
      u = jnp.dot(rbs[m // 2], r1_ref[m], preferred_element_type=jnp.float32)
      u = u * (1.0 / (1.0 + jnp.exp(-u)))
      out_ref[m] = jnp.dot(u, r2_ref[m], preferred_element_type=jnp.float32)

  return pl.pallas_call(
      body,
      grid=(E // BE,),
      in_specs=[
          pl.BlockSpec((BE, D), lambda i: (i, 0)),
          pl.BlockSpec((BE, D), lambda i: (i, 0)),
          pl.BlockSpec((6, NB, RN), lambda i: (0, 0, 0)),
          pl.BlockSpec((6, RN, D), lambda i: (0, 0, 0)),
      ],
      out_specs=pl.BlockSpec((6, BE, D), lambda i: (0, i, 0)),
      out_shape=jax.ShapeDtypeStruct((6, E, D), jnp.float32),
  )


BN = 1000


def _make_tc_node(mode):
  def body(*refs):
    if mode == "out_avg":
      (ha_ref, a0_ref, a1_ref, z_ref, ws_ref, wm_ref, ze_ref, wo_ref,
       prev_ref, out_ref) = refs
    elif mode == "mid":
      ha_ref, a0_ref, a1_ref, z_ref, ws_ref, wm_ref, ze_ref, out_ref = refs
    else:
      (ha_ref, a0_ref, a1_ref, z_ref, ws_ref, wm_ref, ze_ref, wo_ref,
       out_ref) = refs
    h = ha_ref[...]
    agg = (a0_ref[...] + a1_ref[...]) * np.float32(1.0 / np.sqrt(NNEI))
    z = z_ref[...]
    spec = lax.broadcasted_iota(jnp.int32, (BN, NSPEC), 1)
    oneh = (z == spec).astype(jnp.float32)
    acc = (jnp.dot(h, ws_ref[...], preferred_element_type=jnp.float32)
           + jnp.dot(agg, wm_ref[...], preferred_element_type=jnp.float32)
           + jnp.dot(oneh, ze_ref[...], preferred_element_type=jnp.float32))
    hn = acc * (1.0 / (1.0 + jnp.exp(-acc)))
    if mode == "mid":
      out_ref[...] = hn
      return
    out = jnp.dot(hn, wo_ref[...], preferred_element_type=jnp.float32)
    if mode == "out_avg":
      out = (out + prev_ref[...]) * 0.5
    elif mode == "out_abs":
      out = jnp.abs(out)
    out_ref[...] = out

  nd = pl.BlockSpec((BN, D), lambda i: (i, 0))
  dd = pl.BlockSpec((D, D), lambda i: (0, 0))
  in_specs = [nd, nd, nd,
              pl.BlockSpec((BN, 1), lambda i: (i, 0)),
              dd, dd,
              pl.BlockSpec((NSPEC, D), lambda i: (0, 0))]
  if mode != "mid":
    in_specs.append(dd)
  if mode == "out_avg":
    in_specs.append(nd)
  return pl.pallas_call(
      body,
      grid=(N // BN,),
      in_specs=in_specs,
      out_specs=nd,
      out_shape=jax.ShapeDtypeStruct((N, D), jnp.float32),
  )


_gather_pos2 = _make_sc_gather2()
_fused_k = [_make_sc_fused(k) for k in range(6)]
_wall_k = _make_tc_wall()
_node_mid_k = _make_tc_node("mid")
_node_out_k = _make_tc_node("out")
_node_out_avg_k = _make_tc_node("out_avg")
_node_out_abs_k = _make_tc_node("out_abs")


def kernel(pos, x, pos_final_state, x_final_state,
           pos_interpolated_transition_state, species_initial_state,
           species_final_state, batch, edge_index, Wself, Wmsg, R1, R2, Wout,
           Zemb):
  postab = jnp.concatenate(
      [pos, pos_final_state, pos_interpolated_transition_state,
       jnp.zeros((N, D - 9), jnp.float32)], axis=1)
  src_i = edge_index[0].astype(jnp.int32)
  dst_i = edge_index[1].astype(jnp.int32)
  src3 = src_i.reshape(NW, NCHUNK, CH)
  dst3 = dst_i.reshape(NW, NCHUNK, CH)
  src4 = src_i.reshape(NW, NWIN, WWIN, CH2)
  dst4 = dst_i.reshape(NW, NWIN, WWIN, CH2)
  z_init = species_initial_state.astype(jnp.int32).reshape(N, 1)
  z_final = species_final_state.astype(jnp.int32).reshape(N, 1)
  zeros_nd = jnp.zeros((N, D), jnp.float32)
  zeros_pad = jnp.zeros((NPAD, D), jnp.float32)

  possrc, posdst = _gather_pos2(postab, src3, dst3)
  r1all = R1.reshape(6, NB, RN)
  r2all = R2.reshape(6, RN, D)
  wall = _wall_k(possrc, posdst, r1all, r2all).reshape(6 * E, D)

  def seg(which, h):
    return _fused_k[which](wall, h, src4, dst4, zeros_pad)[:, :N]

  a00 = seg(0, x)
  a10 = seg(2, x_final_state)
  h0 = _node_mid_k(x, a00[0], a00[1], z_init, Wself[0, 0], Wmsg[0, 0],
                   Zemb[0])
  a01 = seg(1, h0)
  h1 = _node_mid_k(x_final_state, a10[0], a10[1], z_final, Wself[1, 0],
                   Wmsg[1, 0], Zemb[1])
  a11 = seg(3, h1)
  out_init = _node_out_k(h0, a01[0], a01[1], z_init, Wself[0, 1], Wmsg[0, 1],
                         Zemb[0], Wout[0])
  x_ts = _node_out_avg_k(h1, a11[0], a11[1], z_final, Wself[1, 1],
                         Wmsg[1, 1], Zemb[1], Wout[1], out_init)
  a20 = seg(4, x_ts)
  h2 = _node_mid_k(x_ts, a20[0], a20[1], z_init, Wself[2, 0], Wmsg[2, 0],
                   Zemb[2])
  a21 = seg(5, h2)
  return _node_out_abs_k(h2, a21[0], a21[1], z_init, Wself[2, 1],
                         Wmsg[2, 1], Zemb[2], Wout[2])


# speedup vs baseline: 2.9273x; 1.0314x over previous
import functools

import jax
import jax.numpy as jnp
import numpy as np
from jax import lax
from jax.experimental import pallas as pl
from jax.experimental.pallas import tpu as pltpu
from jax.experimental.pallas import tpu_sc as plsc

N = 10000
E = 320000
D = 128
NB = 16
RN = 64
MAXR = 5.0
NNEI = 32.0
NSPEC = 10

NC = 2
NS = 16
NW = NC * NS
EW = E // NW
CH = 80
NCHUNK = EW // CH
NPAD = 10240
ROWS_PER_TILE = NPAD // NS

_mesh = plsc.VectorSubcoreMesh(
    core_axis_name="c", subcore_axis_name="s", num_cores=NC, num_subcores=NS)


def _worker_id():
  return lax.axis_index("c") * NS + lax.axis_index("s")


def _make_sc_gather2():
  @functools.partial(
      pl.kernel,
      out_type=(jax.ShapeDtypeStruct((E, D), jnp.float32),
                jax.ShapeDtypeStruct((E, D), jnp.float32)),
      mesh=_mesh,
      scratch_types=[
          pltpu.VMEM((NCHUNK, CH), jnp.int32),
          pltpu.VMEM((NCHUNK, CH), jnp.int32),
          pltpu.VMEM((CH, D), jnp.float32),
          pltpu.VMEM((CH, D), jnp.float32),
          pltpu.VMEM((CH, D), jnp.float32),
          pltpu.VMEM((CH, D), jnp.float32),
          pltpu.SemaphoreType.DMA,
          pltpu.SemaphoreType.DMA,
      ],
  )
  def gather_kernel(table_hbm, sidx_hbm, didx_hbm, outs_hbm, outd_hbm,
                    sidx_v, didx_v, sbuf0, sbuf1, dbuf0, dbuf1, sem0, sem1):
    wid = _worker_id()
    pltpu.sync_copy(sidx_hbm.at[wid], sidx_v)
    pltpu.sync_copy(didx_hbm.at[wid], didx_v)
    ebase = wid * EW
    sbufs = (sbuf0, sbuf1)
    dbufs = (dbuf0, dbuf1)
    sems = (sem0, sem1)

    def issue(j, p):
      pltpu.async_copy(table_hbm.at[sidx_v.at[j]], sbufs[p], sems[p])
      pltpu.async_copy(table_hbm.at[didx_v.at[j]], dbufs[p], sems[p])

    def drain(j, p):
      pltpu.make_async_copy(table_hbm.at[sidx_v.at[j]], sbufs[p],
                            sems[p]).wait()
      pltpu.make_async_copy(table_hbm.at[didx_v.at[j]], dbufs[p],
                            sems[p]).wait()
      pltpu.sync_copy(sbufs[p], outs_hbm.at[pl.ds(ebase + j * CH, CH)])
      pltpu.sync_copy(dbufs[p], outd_hbm.at[pl.ds(ebase + j * CH, CH)])

    issue(0, 0)

    def body(j, _):
      for p in range(2):
        jj = 2 * j + p
        issue(jj + 1, 1 - p)
        drain(jj, p)
      return 0

    lax.fori_loop(0, (NCHUNK - 1) // 2, body, 0)
    drain(NCHUNK - 1, 0)

  return gather_kernel


CH2 = 40
NCH2 = EW // CH2
WWIN = 50
NWIN = NCH2 // WWIN


def _make_sc_fused(which):
  wbase0 = which * E

  @functools.partial(
      pl.kernel,
      out_type=jax.ShapeDtypeStruct((NC, NPAD, D), jnp.float32),
      mesh=_mesh,
      scratch_types=[
          pltpu.VMEM((WWIN, CH2), jnp.int32),
          pltpu.VMEM((WWIN, CH2), jnp.int32),
          pltpu.VMEM((CH2, D), jnp.float32),
          pltpu.VMEM((CH2, D), jnp.float32),
          pltpu.VMEM((CH2, D), jnp.float32),
          pltpu.VMEM((CH2, D), jnp.float32),
          pltpu.VMEM((CH2, D), jnp.float32),
          pltpu.VMEM((CH2, D), jnp.float32),
          pltpu.VMEM_SHARED((NPAD, D), jnp.float32),
          pltpu.SemaphoreType.DMA,
          pltpu.SemaphoreType.DMA,
          pltpu.SemaphoreType.DMA,
          pltpu.SemaphoreType.DMA,
      ],
  )
  def fused_kernel(w6_hbm, h_hbm, src_hbm, dst_hbm, zeros_hbm, out_hbm,
                   src_v, dst_v, wb0, wb1, hb0, hb1, mb0, mb1, acc_sh,
                   sem0, sem1, scs0, scs1):
    c = lax.axis_index("c")
    s = lax.axis_index("s")
    wid = c * NS + s
    rbase = s * ROWS_PER_TILE
    pltpu.sync_copy(zeros_hbm.at[pl.ds(rbase, ROWS_PER_TILE)],
                    acc_sh.at[pl.ds(rbase, ROWS_PER_TILE)])
    plsc.subcore_barrier()
    ebase = wid * EW
    wbase = wbase0 + ebase
    wbufs = (wb0, wb1)
    hbufs = (hb0, hb1)
    mbufs = (mb0, mb1)
    sems = (sem0, sem1)
    scsems = (scs0, scs1)

    def win_body(win, _):
      pltpu.sync_copy(src_hbm.at[wid, win], src_v)
      pltpu.sync_copy(dst_hbm.at[wid, win], dst_v)
      wb_e = wbase + win * (WWIN * CH2)

      def issue(j, p):
        pltpu.async_copy(w6_hbm.at[pl.ds(wb_e + j * CH2, CH2)], wbufs[p],
                         sems[p])
        pltpu.async_copy(h_hbm.at[src_v.at[j]], hbufs[p], sems[p])

      def wait(j, p):
        pltpu.make_async_copy(w6_hbm.at[pl.ds(wb_e + j * CH2, CH2)],
                              wbufs[p], sems[p]).wait()
        pltpu.make_async_copy(h_hbm.at[src_v.at[j]], hbufs[p],
                              sems[p]).wait()

      def wait_sc(j, p):
        pltpu.make_async_copy(mbufs[p], acc_sh.at[dst_v.at[j]],
                              scsems[p]).wait()

      def process(j, p):
        wb = wbufs[p]
        hb = hbufs[p]
        mb = mbufs[p]

        def mrow(i, _):
          for k in range(D // 16):
            sl = pl.ds(k * 16, 16)
            mb[i, sl] = wb[i, sl] * hb[i, sl]
          return 0

        lax.fori_loop(0, CH2, mrow, 0)
        pltpu.async_copy(mb, acc_sh.at[dst_v.at[j]], scsems[p], add=True)

      issue(0, 0)

      def pair(i, _):
        j0 = 2 * i
        issue(j0 + 1, 1)
        wait(j0, 0)

        @pl.when(i > 0)
        def _():
          wait_sc(j0 - 2, 0)

        process(j0, 0)

        @pl.when(j0 + 2 < WWIN)
        def _():
          issue(j0 + 2, 0)

        wait(j0 + 1, 1)

        @pl.when(i > 0)
        def _():
          wait_sc(j0 - 1, 1)

        process(j0 + 1, 1)
        return 0

      lax.fori_loop(0, WWIN // 2, pair, 0)
      wait_sc(WWIN - 2, 0)
      wait_sc(WWIN - 1, 1)
      return 0

    lax.fori_loop(0, NWIN, win_body, 0)
    plsc.subcore_barrier()
    pltpu.sync_copy(acc_sh.at[pl.ds(rbase, ROWS_PER_TILE)],
                    out_hbm.at[c, pl.ds(rbase, ROWS_PER_TILE)])

  return fused_kernel


BE = 2000
_SIG = MAXR / NB
_INV2SIG2 = np.float32(1.0 / (2.0 * _SIG * _SIG))
_CSTEP = np.float32(MAXR / (NB - 1))


def _make_tc_wall():
  def body(ps_ref, pd_ref, r1_ref, r2_ref, out_ref):
    col = lax.broadcasted_iota(jnp.int32, (1, D), 1)
    centers = lax.broadcasted_iota(jnp.int32, (1, NB), 1).astype(
        jnp.float32) * _CSTEP
    diff = ps_ref[...] - pd_ref[...]
    dsq = diff * diff
    for net in range(3):
      mask = ((col >= 3 * net) & (col < 3 * net + 3)).astype(jnp.float32)
      d2 = jnp.sum(dsq * mask, axis=1, keepdims=True) + 1e-12
      r = jnp.sqrt(d2)
      rb = jnp.exp(-((r - centers) ** 2) * _INV2SIG2)
      cut = 0.5 * (jnp.cos(jnp.pi * jnp.clip(r * (1.0 / MAXR), 0.0, 1.0))
                   + 1.0)
      rb = (rb * cut).astype(jnp.bfloat16)
      u = jnp.dot(rb, r1_ref[net], preferred_element_type=jnp.float32)
      u = (u * (1.0 / (1.0 + jnp.exp(-u)))).astype(jnp.bfloat16)
      w2 = jnp.dot(u, r2_ref[net], preferred_element_type=jnp.float32)
      out_ref[2 * net] = w2[:, :D]
      out_ref[2 * net + 1] = w2[:, D:]

  return pl.pallas_call(
      body,
      grid=(E // BE,),
      in_specs=[
          pl.BlockSpec((BE, D), lambda i: (i, 0)),
          pl.BlockSpec((BE, D), lambda i: (i, 0)),
          pl.BlockSpec((3, NB, 2 * RN), lambda i: (0, 0, 0)),
          pl.BlockSpec((3, 2 * RN, 2 * D), lambda i: (0, 0, 0)),
      ],
      out_specs=pl.BlockSpec((6, BE, D), lambda i: (0, i, 0)),
      out_shape=jax.ShapeDtypeStruct((6, E, D), jnp.float32),
  )


BN = 1000


def _make_tc_node(mode):
  def body(*refs):
    if mode == "out_avg":
      (ha_ref, a0_ref, a1_ref, z_ref, ws_ref, wm_ref, ze_ref, wo_ref,
       prev_ref, out_ref) = refs
    elif mode == "mid":
      ha_ref, a0_ref, a1_ref, z_ref, ws_ref, wm_ref, ze_ref, out_ref = refs
    else:
      (ha_ref, a0_ref, a1_ref, z_ref, ws_ref, wm_ref, ze_ref, wo_ref,
       out_ref) = refs
    h = ha_ref[...]
    agg = (a0_ref[...] + a1_ref[...]) * np.float32(1.0 / np.sqrt(NNEI))
    z = z_ref[...]
    spec = lax.broadcasted_iota(jnp.int32, (BN, NSPEC), 1)
    oneh = (z == spec).astype(jnp.float32)
    acc = (jnp.dot(h, ws_ref[...], preferred_element_type=jnp.float32)
           + jnp.dot(agg, wm_ref[...], preferred_element_type=jnp.float32)
           + jnp.dot(oneh, ze_ref[...], preferred_element_type=jnp.float32))
    hn = acc * (1.0 / (1.0 + jnp.exp(-acc)))
    if mode == "mid":
      out_ref[...] = hn
      return
    out = jnp.dot(hn, wo_ref[...], preferred_element_type=jnp.float32)
    if mode == "out_avg":
      out = (out + prev_ref[...]) * 0.5
    elif mode == "out_abs":
      out = jnp.abs(out)
    out_ref[...] = out

  nd = pl.BlockSpec((BN, D), lambda i: (i, 0))
  dd = pl.BlockSpec((D, D), lambda i: (0, 0))
  in_specs = [nd, nd, nd,
              pl.BlockSpec((BN, 1), lambda i: (i, 0)),
              dd, dd,
              pl.BlockSpec((NSPEC, D), lambda i: (0, 0))]
  if mode != "mid":
    in_specs.append(dd)
  if mode == "out_avg":
    in_specs.append(nd)
  return pl.pallas_call(
      body,
      grid=(N // BN,),
      in_specs=in_specs,
      out_specs=nd,
      out_shape=jax.ShapeDtypeStruct((N, D), jnp.float32),
  )


_gather_pos2 = _make_sc_gather2()
_fused_k = [_make_sc_fused(k) for k in range(6)]
_wall_k = _make_tc_wall()
_node_mid_k = _make_tc_node("mid")
_node_out_k = _make_tc_node("out")
_node_out_avg_k = _make_tc_node("out_avg")
_node_out_abs_k = _make_tc_node("out_abs")


def kernel(pos, x, pos_final_state, x_final_state,
           pos_interpolated_transition_state, species_initial_state,
           species_final_state, batch, edge_index, Wself, Wmsg, R1, R2, Wout,
           Zemb):
  postab = jnp.concatenate(
      [pos, pos_final_state, pos_interpolated_transition_state,
       jnp.zeros((N, D - 9), jnp.float32)], axis=1)
  src_i = edge_index[0].astype(jnp.int32)
  dst_i = edge_index[1].astype(jnp.int32)
  src3 = src_i.reshape(NW, NCHUNK, CH)
  dst3 = dst_i.reshape(NW, NCHUNK, CH)
  src4 = src_i.reshape(NW, NWIN, WWIN, CH2)
  dst4 = dst_i.reshape(NW, NWIN, WWIN, CH2)
  z_init = species_initial_state.astype(jnp.int32).reshape(N, 1)
  z_final = species_final_state.astype(jnp.int32).reshape(N, 1)
  zeros_nd = jnp.zeros((N, D), jnp.float32)
  zeros_pad = jnp.zeros((NPAD, D), jnp.float32)

  possrc, posdst = _gather_pos2(postab, src3, dst3)
  r1cat = jnp.concatenate([R1[:, 0], R1[:, 1]], axis=2).astype(jnp.bfloat16)
  zblk = jnp.zeros((3, RN, D), jnp.float32)
  r2bd = jnp.concatenate(
      [jnp.concatenate([R2[:, 0], zblk], axis=2),
       jnp.concatenate([zblk, R2[:, 1]], axis=2)],
      axis=1).astype(jnp.bfloat16)
  wall = _wall_k(possrc, posdst, r1cat, r2bd).reshape(6 * E, D)

  def seg(which, h):
    return _fused_k[which](wall, h, src4, dst4, zeros_pad)[:, :N]

  a00 = seg(0, x)
  a10 = seg(2, x_final_state)
  h0 = _node_mid_k(x, a00[0], a00[1], z_init, Wself[0, 0], Wmsg[0, 0],
                   Zemb[0])
  a01 = seg(1, h0)
  h1 = _node_mid_k(x_final_state, a10[0], a10[1], z_final, Wself[1, 0],
                   Wmsg[1, 0], Zemb[1])
  a11 = seg(3, h1)
  out_init = _node_out_k(h0, a01[0], a01[1], z_init, Wself[0, 1], Wmsg[0, 1],
                         Zemb[0], Wout[0])
  x_ts = _node_out_avg_k(h1, a11[0], a11[1], z_final, Wself[1, 1],
                         Wmsg[1, 1], Zemb[1], Wout[1], out_init)
  a20 = seg(4, x_ts)
  h2 = _node_mid_k(x_ts, a20[0], a20[1], z_init, Wself[2, 0], Wmsg[2, 0],
                   Zemb[2])
  a21 = seg(5, h2)
  return _node_out_abs_k(h2, a21[0], a21[1], z_init, Wself[2, 1],
                         Wmsg[2, 1], Zemb[2], Wout[2])


# speedup vs baseline: 4.9149x; 1.6790x over previous
import functools

import jax
import jax.numpy as jnp
import numpy as np
from jax import lax
from jax.experimental import pallas as pl
from jax.experimental.pallas import tpu as pltpu
from jax.experimental.pallas import tpu_sc as plsc

N = 10000
E = 320000
D = 128
NB = 16
RN = 64
MAXR = 5.0
NNEI = 32.0
NSPEC = 10

NC = 2
NS = 16
NW = NC * NS
EW = E // NW
CH = 80
NCHUNK = EW // CH
NPAD = 10240
ROWS_PER_TILE = NPAD // NS

_mesh = plsc.VectorSubcoreMesh(
    core_axis_name="c", subcore_axis_name="s", num_cores=NC, num_subcores=NS)


def _worker_id():
  return lax.axis_index("c") * NS + lax.axis_index("s")


def _make_sc_gather2():
  @functools.partial(
      pl.kernel,
      out_type=(jax.ShapeDtypeStruct((E, D), jnp.float32),
                jax.ShapeDtypeStruct((E, D), jnp.float32)),
      mesh=_mesh,
      scratch_types=[
          pltpu.VMEM((NCHUNK, CH), jnp.int32),
          pltpu.VMEM((NCHUNK, CH), jnp.int32),
          pltpu.VMEM((CH, D), jnp.float32),
          pltpu.VMEM((CH, D), jnp.float32),
          pltpu.VMEM((CH, D), jnp.float32),
          pltpu.VMEM((CH, D), jnp.float32),
          pltpu.SemaphoreType.DMA,
          pltpu.SemaphoreType.DMA,
      ],
  )
  def gather_kernel(table_hbm, sidx_hbm, didx_hbm, outs_hbm, outd_hbm,
                    sidx_v, didx_v, sbuf0, sbuf1, dbuf0, dbuf1, sem0, sem1):
    wid = _worker_id()
    pltpu.sync_copy(sidx_hbm.at[wid], sidx_v)
    pltpu.sync_copy(didx_hbm.at[wid], didx_v)
    ebase = wid * EW
    sbufs = (sbuf0, sbuf1)
    dbufs = (dbuf0, dbuf1)
    sems = (sem0, sem1)

    def issue(j, p):
      pltpu.async_copy(table_hbm.at[sidx_v.at[j]], sbufs[p], sems[p])
      pltpu.async_copy(table_hbm.at[didx_v.at[j]], dbufs[p], sems[p])

    def drain(j, p):
      pltpu.make_async_copy(table_hbm.at[sidx_v.at[j]], sbufs[p],
                            sems[p]).wait()
      pltpu.make_async_copy(table_hbm.at[didx_v.at[j]], dbufs[p],
                            sems[p]).wait()
      pltpu.sync_copy(sbufs[p], outs_hbm.at[pl.ds(ebase + j * CH, CH)])
      pltpu.sync_copy(dbufs[p], outd_hbm.at[pl.ds(ebase + j * CH, CH)])

    issue(0, 0)

    def body(j, _):
      for p in range(2):
        jj = 2 * j + p
        issue(jj + 1, 1 - p)
        drain(jj, p)
      return 0

    lax.fori_loop(0, (NCHUNK - 1) // 2, body, 0)
    drain(NCHUNK - 1, 0)

  return gather_kernel


CH2 = 40
NCH2 = EW // CH2
WWIN = 50
NWIN = NCH2 // WWIN


def _make_sc_fused(which):
  wbase0 = which * E

  @functools.partial(
      pl.kernel,
      out_type=jax.ShapeDtypeStruct((NC, NPAD, D), jnp.float32),
      mesh=_mesh,
      scratch_types=[
          pltpu.VMEM((WWIN, CH2), jnp.int32),
          pltpu.VMEM((WWIN, CH2), jnp.int32),
          pltpu.VMEM((CH2, D), jnp.float32),
          pltpu.VMEM((CH2, D), jnp.float32),
          pltpu.VMEM((CH2, D), jnp.float32),
          pltpu.VMEM((CH2, D), jnp.float32),
          pltpu.VMEM((CH2, D), jnp.float32),
          pltpu.VMEM((CH2, D), jnp.float32),
          pltpu.VMEM_SHARED((NPAD, D), jnp.float32),
          pltpu.SemaphoreType.DMA,
          pltpu.SemaphoreType.DMA,
          pltpu.SemaphoreType.DMA,
          pltpu.SemaphoreType.DMA,
      ],
  )
  def fused_kernel(w6_hbm, h_hbm, src_hbm, dst_hbm, zeros_hbm, out_hbm,
                   src_v, dst_v, wb0, wb1, hb0, hb1, mb0, mb1, acc_sh,
                   sem0, sem1, scs0, scs1):
    c = lax.axis_index("c")
    s = lax.axis_index("s")
    wid = c * NS + s
    rbase = s * ROWS_PER_TILE
    pltpu.sync_copy(zeros_hbm.at[pl.ds(rbase, ROWS_PER_TILE)],
                    acc_sh.at[pl.ds(rbase, ROWS_PER_TILE)])
    plsc.subcore_barrier()
    ebase = wid * EW
    wbase = wbase0 + ebase
    wbufs = (wb0, wb1)
    hbufs = (hb0, hb1)
    mbufs = (mb0, mb1)
    sems = (sem0, sem1)
    scsems = (scs0, scs1)

    def win_body(win, _):
      pltpu.sync_copy(src_hbm.at[wid, win], src_v)
      pltpu.sync_copy(dst_hbm.at[wid, win], dst_v)
      wb_e = wbase + win * (WWIN * CH2)

      def issue(j, p):
        pltpu.async_copy(w6_hbm.at[pl.ds(wb_e + j * CH2, CH2)], wbufs[p],
                         sems[p])
        pltpu.async_copy(h_hbm.at[src_v.at[j]], hbufs[p], sems[p])

      def wait(j, p):
        pltpu.make_async_copy(w6_hbm.at[pl.ds(wb_e + j * CH2, CH2)],
                              wbufs[p], sems[p]).wait()
        pltpu.make_async_copy(h_hbm.at[src_v.at[j]], hbufs[p],
                              sems[p]).wait()

      def wait_sc(j, p):
        pltpu.make_async_copy(mbufs[p], acc_sh.at[dst_v.at[j]],
                              scsems[p]).wait()

      def process(j, p):
        wb = wbufs[p]
        hb = hbufs[p]
        mb = mbufs[p]

        def mrow(i, _):
          for k in range(D // 16):
            sl = pl.ds(k * 16, 16)
            mb[i, sl] = wb[i, sl] * hb[i, sl]
          return 0

        lax.fori_loop(0, CH2, mrow, 0)
        pltpu.async_copy(mb, acc_sh.at[dst_v.at[j]], scsems[p], add=True)

      issue(0, 0)

      def pair(i, _):
        j0 = 2 * i
        issue(j0 + 1, 1)
        wait(j0, 0)

        @pl.when(i > 0)
        def _():
          wait_sc(j0 - 2, 0)

        process(j0, 0)

        @pl.when(j0 + 2 < WWIN)
        def _():
          issue(j0 + 2, 0)

        wait(j0 + 1, 1)

        @pl.when(i > 0)
        def _():
          wait_sc(j0 - 1, 1)

        process(j0 + 1, 1)
        return 0

      lax.fori_loop(0, WWIN // 2, pair, 0)
      wait_sc(WWIN - 2, 0)
      wait_sc(WWIN - 1, 1)
      return 0

    lax.fori_loop(0, NWIN, win_body, 0)
    plsc.subcore_barrier()
    pltpu.sync_copy(acc_sh.at[pl.ds(rbase, ROWS_PER_TILE)],
                    out_hbm.at[c, pl.ds(rbase, ROWS_PER_TILE)])

  return fused_kernel


BE = 2000
_SIG = MAXR / NB
_INV2SIG2 = np.float32(1.0 / (2.0 * _SIG * _SIG))
_CSTEP = np.float32(MAXR / (NB - 1))


def _make_tc_wall():
  def body(ps_ref, pd_ref, r1_ref, r2_ref, out_ref):
    centers = lax.broadcasted_iota(jnp.int32, (1, NB), 1).astype(
        jnp.float32) * _CSTEP
    diff = ps_ref[:, :NB] - pd_ref[:, :NB]
    dsq = diff * diff
    d2all = dsq[:, 0:3] + dsq[:, 3:6] + dsq[:, 6:9] + 1e-12
    rall = jnp.sqrt(d2all)
    t = jnp.minimum(rall * (1.0 / MAXR), 1.0)
    u = t * t
    cosp = np.float32(0.0016053627762021867)
    for cc in (-0.025391111383297586, 0.2350633717621909,
               -1.3351744534102399, 4.0586982622690035,
               -4.934801388370911, 0.9999999922898454):
      cosp = cosp * u + np.float32(cc)
    cutall = 0.5 * (cosp + 1.0)
    for net in range(3):
      r = rall[:, net:net + 1]
      cut = cutall[:, net:net + 1]
      rb = jnp.exp(-((r - centers) ** 2) * _INV2SIG2)
      rb = (rb * cut).astype(jnp.bfloat16)
      u = jnp.dot(rb, r1_ref[net], preferred_element_type=jnp.float32)
      u = (u * (1.0 / (1.0 + jnp.exp(-u)))).astype(jnp.bfloat16)
      w2 = jnp.dot(u, r2_ref[net], preferred_element_type=jnp.float32)
      out_ref[2 * net] = w2[:, :D]
      out_ref[2 * net + 1] = w2[:, D:]

  return pl.pallas_call(
      body,
      grid=(E // BE,),
      in_specs=[
          pl.BlockSpec((BE, D), lambda i: (i, 0)),
          pl.BlockSpec((BE, D), lambda i: (i, 0)),
          pl.BlockSpec((3, NB, 2 * RN), lambda i: (0, 0, 0)),
          pl.BlockSpec((3, 2 * RN, 2 * D), lambda i: (0, 0, 0)),
      ],
      out_specs=pl.BlockSpec((6, BE, D), lambda i: (0, i, 0)),
      out_shape=jax.ShapeDtypeStruct((6, E, D), jnp.float32),
  )


BN = 1000


def _make_tc_node(mode):
  def body(*refs):
    if mode == "out_avg":
      (ha_ref, a0_ref, a1_ref, z_ref, ws_ref, wm_ref, ze_ref, wo_ref,
       prev_ref, out_ref) = refs
    elif mode == "mid":
      ha_ref, a0_ref, a1_ref, z_ref, ws_ref, wm_ref, ze_ref, out_ref = refs
    else:
      (ha_ref, a0_ref, a1_ref, z_ref, ws_ref, wm_ref, ze_ref, wo_ref,
       out_ref) = refs
    h = ha_ref[...]
    agg = (a0_ref[...] + a1_ref[...]) * np.float32(1.0 / np.sqrt(NNEI))
    z = z_ref[...]
    spec = lax.broadcasted_iota(jnp.int32, (BN, NSPEC), 1)
    oneh = (z == spec).astype(jnp.float32)
    acc = (jnp.dot(h, ws_ref[...], preferred_element_type=jnp.float32)
           + jnp.dot(agg, wm_ref[...], preferred_element_type=jnp.float32)
           + jnp.dot(oneh, ze_ref[...], preferred_element_type=jnp.float32))
    hn = acc * (1.0 / (1.0 + jnp.exp(-acc)))
    if mode == "mid":
      out_ref[...] = hn
      return
    out = jnp.dot(hn, wo_ref[...], preferred_element_type=jnp.float32)
    if mode == "out_avg":
      out = (out + prev_ref[...]) * 0.5
    elif mode == "out_abs":
      out = jnp.abs(out)
    out_ref[...] = out

  nd = pl.BlockSpec((BN, D), lambda i: (i, 0))
  dd = pl.BlockSpec((D, D), lambda i: (0, 0))
  in_specs = [nd, nd, nd,
              pl.BlockSpec((BN, 1), lambda i: (i, 0)),
              dd, dd,
              pl.BlockSpec((NSPEC, D), lambda i: (0, 0))]
  if mode != "mid":
    in_specs.append(dd)
  if mode == "out_avg":
    in_specs.append(nd)
  return pl.pallas_call(
      body,
      grid=(N // BN,),
      in_specs=in_specs,
      out_specs=nd,
      out_shape=jax.ShapeDtypeStruct((N, D), jnp.float32),
  )


_gather_pos2 = _make_sc_gather2()
_fused_k = [_make_sc_fused(k) for k in range(6)]
_wall_k = _make_tc_wall()
_node_mid_k = _make_tc_node("mid")
_node_out_k = _make_tc_node("out")
_node_out_avg_k = _make_tc_node("out_avg")
_node_out_abs_k = _make_tc_node("out_abs")


def kernel(pos, x, pos_final_state, x_final_state,
           pos_interpolated_transition_state, species_initial_state,
           species_final_state, batch, edge_index, Wself, Wmsg, R1, R2, Wout,
           Zemb):
  postab = jnp.concatenate(
      [pos, pos_final_state, pos_interpolated_transition_state,
       jnp.zeros((N, D - 9), jnp.float32)], axis=1)
  src_i = edge_index[0].astype(jnp.int32)
  dst_i = edge_index[1].astype(jnp.int32)
  src3 = src_i.reshape(NW, NCHUNK, CH)
  dst3 = dst_i.reshape(NW, NCHUNK, CH)
  src4 = src_i.reshape(NW, NWIN, WWIN, CH2)
  dst4 = dst_i.reshape(NW, NWIN, WWIN, CH2)
  z_init = species_initial_state.astype(jnp.int32).reshape(N, 1)
  z_final = species_final_state.astype(jnp.int32).reshape(N, 1)
  zeros_nd = jnp.zeros((N, D), jnp.float32)
  zeros_pad = jnp.zeros((NPAD, D), jnp.float32)

  possrc, posdst = _gather_pos2(postab, src3, dst3)
  r1cat = jnp.concatenate([R1[:, 0], R1[:, 1]], axis=2).astype(jnp.bfloat16)
  zblk = jnp.zeros((3, RN, D), jnp.float32)
  r2bd = jnp.concatenate(
      [jnp.concatenate([R2[:, 0], zblk], axis=2),
       jnp.concatenate([zblk, R2[:, 1]], axis=2)],
      axis=1).astype(jnp.bfloat16)
  wall = _wall_k(possrc, posdst, r1cat, r2bd).reshape(6 * E, D)

  def seg(which, h):
    return _fused_k[which](wall, h, src4, dst4, zeros_pad)[:, :N]

  a00 = seg(0, x)
  a10 = seg(2, x_final_state)
  h0 = _node_mid_k(x, a00[0], a00[1], z_init, Wself[0, 0], Wmsg[0, 0],
                   Zemb[0])
  a01 = seg(1, h0)
  h1 = _node_mid_k(x_final_state, a10[0], a10[1], z_final, Wself[1, 0],
                   Wmsg[1, 0], Zemb[1])
  a11 = seg(3, h1)
  out_init = _node_out_k(h0, a01[0], a01[1], z_init, Wself[0, 1], Wmsg[0, 1],
                         Zemb[0], Wout[0])
  x_ts = _node_out_avg_k(h1, a11[0], a11[1], z_final, Wself[1, 1],
                         Wmsg[1, 1], Zemb[1], Wout[1], out_init)
  a20 = seg(4, x_ts)
  h2 = _node_mid_k(x_ts, a20[0], a20[1], z_init, Wself[2, 0], Wmsg[2, 0],
                   Zemb[2])
  a21 = seg(5, h2)
  return _node_out_abs_k(h2, a21[0], a21[1], z_init, Wself[2, 1],
                         Wmsg[2, 1], Zemb[2], Wout[2])


# speedup vs baseline: 4.9814x; 1.0135x over previous
import functools

import jax
import jax.numpy as jnp
import numpy as np
from jax import lax
from jax.experimental import pallas as pl
from jax.experimental.pallas import tpu as pltpu
from jax.experimental.pallas import tpu_sc as plsc

N = 10000
E = 320000
D = 128
NB = 16
RN = 64
MAXR = 5.0
NNEI = 32.0
NSPEC = 10

NC = 2
NS = 16
NW = NC * NS
EW = E // NW
CH = 80
NCHUNK = EW // CH
NPAD = 10240
ROWS_PER_TILE = NPAD // NS

_mesh = plsc.VectorSubcoreMesh(
    core_axis_name="c", subcore_axis_name="s", num_cores=NC, num_subcores=NS)


def _worker_id():
  return lax.axis_index("c") * NS + lax.axis_index("s")


def _make_sc_gather2():
  @functools.partial(
      pl.kernel,
      out_type=(jax.ShapeDtypeStruct((E, D), jnp.float32),
                jax.ShapeDtypeStruct((E, D), jnp.float32)),
      mesh=_mesh,
      scratch_types=[
          pltpu.VMEM((NCHUNK, CH), jnp.int32),
          pltpu.VMEM((NCHUNK, CH), jnp.int32),
          pltpu.VMEM((CH, D), jnp.float32),
          pltpu.VMEM((CH, D), jnp.float32),
          pltpu.VMEM((CH, D), jnp.float32),
          pltpu.VMEM((CH, D), jnp.float32),
          pltpu.SemaphoreType.DMA,
          pltpu.SemaphoreType.DMA,
      ],
  )
  def gather_kernel(table_hbm, sidx_hbm, didx_hbm, outs_hbm, outd_hbm,
                    sidx_v, didx_v, sbuf0, sbuf1, dbuf0, dbuf1, sem0, sem1):
    wid = _worker_id()
    pltpu.sync_copy(sidx_hbm.at[wid], sidx_v)
    pltpu.sync_copy(didx_hbm.at[wid], didx_v)
    ebase = wid * EW
    sbufs = (sbuf0, sbuf1)
    dbufs = (dbuf0, dbuf1)
    sems = (sem0, sem1)

    def issue(j, p):
      pltpu.async_copy(table_hbm.at[sidx_v.at[j]], sbufs[p], sems[p])
      pltpu.async_copy(table_hbm.at[didx_v.at[j]], dbufs[p], sems[p])

    def drain(j, p):
      pltpu.make_async_copy(table_hbm.at[sidx_v.at[j]], sbufs[p],
                            sems[p]).wait()
      pltpu.make_async_copy(table_hbm.at[didx_v.at[j]], dbufs[p],
                            sems[p]).wait()
      pltpu.sync_copy(sbufs[p], outs_hbm.at[pl.ds(ebase + j * CH, CH)])
      pltpu.sync_copy(dbufs[p], outd_hbm.at[pl.ds(ebase + j * CH, CH)])

    issue(0, 0)

    def body(j, _):
      for p in range(2):
        jj = 2 * j + p
        issue(jj + 1, 1 - p)
        drain(jj, p)
      return 0

    lax.fori_loop(0, (NCHUNK - 1) // 2, body, 0)
    drain(NCHUNK - 1, 0)

  return gather_kernel


CH2 = 40
NCH2 = EW // CH2
WWIN = 50
NWIN = NCH2 // WWIN


def _make_sc_fused(which):
  wbase0 = which * E

  @functools.partial(
      pl.kernel,
      out_type=jax.ShapeDtypeStruct((NC, NPAD, D), jnp.float32),
      mesh=_mesh,
      scratch_types=[
          pltpu.VMEM((WWIN, CH2), jnp.int32),
          pltpu.VMEM((WWIN, CH2), jnp.int32),
          pltpu.VMEM((CH2, D), jnp.float32),
          pltpu.VMEM((CH2, D), jnp.float32),
          pltpu.VMEM((CH2, D), jnp.float32),
          pltpu.VMEM((CH2, D), jnp.float32),
          pltpu.VMEM((CH2, D), jnp.float32),
          pltpu.VMEM((CH2, D), jnp.float32),
          pltpu.VMEM_SHARED((NPAD, D), jnp.float32),
          pltpu.SemaphoreType.DMA,
          pltpu.SemaphoreType.DMA,
          pltpu.SemaphoreType.DMA,
          pltpu.SemaphoreType.DMA,
      ],
  )
  def fused_kernel(w6_hbm, h_hbm, src_hbm, dst_hbm, zeros_hbm, out_hbm,
                   src_v, dst_v, wb0, wb1, hb0, hb1, mb0, mb1, acc_sh,
                   sem0, sem1, scs0, scs1):
    c = lax.axis_index("c")
    s = lax.axis_index("s")
    wid = c * NS + s
    rbase = s * ROWS_PER_TILE
    pltpu.sync_copy(zeros_hbm.at[pl.ds(rbase, ROWS_PER_TILE)],
                    acc_sh.at[pl.ds(rbase, ROWS_PER_TILE)])
    plsc.subcore_barrier()
    ebase = wid * EW
    wbase = wbase0 + ebase
    wbufs = (wb0, wb1)
    hbufs = (hb0, hb1)
    mbufs = (mb0, mb1)
    sems = (sem0, sem1)
    scsems = (scs0, scs1)

    def win_body(win, _):
      pltpu.sync_copy(src_hbm.at[wid, win], src_v)
      pltpu.sync_copy(dst_hbm.at[wid, win], dst_v)
      wb_e = wbase + win * (WWIN * CH2)

      def issue(j, p):
        pltpu.async_copy(w6_hbm.at[pl.ds(wb_e + j * CH2, CH2)], wbufs[p],
                         sems[p])
        pltpu.async_copy(h_hbm.at[src_v.at[j]], hbufs[p], sems[p])

      def wait(j, p):
        pltpu.make_async_copy(w6_hbm.at[pl.ds(wb_e + j * CH2, CH2)],
                              wbufs[p], sems[p]).wait()
        pltpu.make_async_copy(h_hbm.at[src_v.at[j]], hbufs[p],
                              sems[p]).wait()

      def wait_sc(j, p):
        pltpu.make_async_copy(mbufs[p], acc_sh.at[dst_v.at[j]],
                              scsems[p]).wait()

      def process(j, p):
        wb = wbufs[p]
        hb = hbufs[p]
        mb = mbufs[p]

        def mrow(i, _):
          for k in range(D // 16):
            sl = pl.ds(k * 16, 16)
            mb[i, sl] = wb[i, sl] * hb[i, sl]
          return 0

        lax.fori_loop(0, CH2, mrow, 0)
        pltpu.async_copy(mb, acc_sh.at[dst_v.at[j]], scsems[p], add=True)

      issue(0, 0)

      def pair(i, _):
        j0 = 2 * i
        issue(j0 + 1, 1)
        wait(j0, 0)

        @pl.when(i > 0)
        def _():
          wait_sc(j0 - 2, 0)

        process(j0, 0)

        @pl.when(j0 + 2 < WWIN)
        def _():
          issue(j0 + 2, 0)

        wait(j0 + 1, 1)

        @pl.when(i > 0)
        def _():
          wait_sc(j0 - 1, 1)

        process(j0 + 1, 1)
        return 0

      lax.fori_loop(0, WWIN // 2, pair, 0)
      wait_sc(WWIN - 2, 0)
      wait_sc(WWIN - 1, 1)
      return 0

    lax.fori_loop(0, NWIN, win_body, 0)
    plsc.subcore_barrier()
    pltpu.sync_copy(acc_sh.at[pl.ds(rbase, ROWS_PER_TILE)],
                    out_hbm.at[c, pl.ds(rbase, ROWS_PER_TILE)])

  return fused_kernel


BE = 2000
_SIG = MAXR / NB
_INV2SIG2 = np.float32(1.0 / (2.0 * _SIG * _SIG))
_CSTEP = np.float32(MAXR / (NB - 1))


def _make_tc_wall():
  def body(ps_ref, pd_ref, r1_ref, r2_ref, out_ref):
    centers = lax.broadcasted_iota(jnp.int32, (1, NB), 1).astype(
        jnp.float32) * _CSTEP
    diff = ps_ref[:, :NB] - pd_ref[:, :NB]
    dsq = diff * diff
    colsel = lax.broadcasted_iota(jnp.int32, (NB, 3), 0)
    netsel = lax.broadcasted_iota(jnp.int32, (NB, 3), 1)
    sel = ((colsel >= 3 * netsel) & (colsel < 3 * netsel + 3)).astype(
        jnp.float32)
    d2all = jnp.dot(dsq, sel, preferred_element_type=jnp.float32) + 1e-12
    rall = jnp.sqrt(d2all)
    t = jnp.minimum(rall * (1.0 / MAXR), 1.0)
    u = t * t
    cosp = np.float32(0.0016053627762021867)
    for cc in (-0.025391111383297586, 0.2350633717621909,
               -1.3351744534102399, 4.0586982622690035,
               -4.934801388370911, 0.9999999922898454):
      cosp = cosp * u + np.float32(cc)
    cutall = 0.5 * (cosp + 1.0)
    for net in range(3):
      r = rall[:, net:net + 1]
      cut = cutall[:, net:net + 1]
      rb = jnp.exp(-((r - centers) ** 2) * _INV2SIG2)
      rb = (rb * cut).astype(jnp.bfloat16)
      u = jnp.dot(rb, r1_ref[net], preferred_element_type=jnp.float32)
      u = (u * (1.0 / (1.0 + jnp.exp(-u)))).astype(jnp.bfloat16)
      w2 = jnp.dot(u, r2_ref[net], preferred_element_type=jnp.float32)
      out_ref[2 * net] = w2[:, :D]
      out_ref[2 * net + 1] = w2[:, D:]

  return pl.pallas_call(
      body,
      grid=(E // BE,),
      in_specs=[
          pl.BlockSpec((BE, D), lambda i: (i, 0)),
          pl.BlockSpec((BE, D), lambda i: (i, 0)),
          pl.BlockSpec((3, NB, 2 * RN), lambda i: (0, 0, 0)),
          pl.BlockSpec((3, 2 * RN, 2 * D), lambda i: (0, 0, 0)),
      ],
      out_specs=pl.BlockSpec((6, BE, D), lambda i: (0, i, 0)),
      out_shape=jax.ShapeDtypeStruct((6, E, D), jnp.float32),
  )


BN = 1000


def _make_tc_node(mode):
  def body(*refs):
    if mode == "out_avg":
      (ha_ref, a0_ref, a1_ref, z_ref, ws_ref, wm_ref, ze_ref, wo_ref,
       prev_ref, out_ref) = refs
    elif mode == "mid":
      ha_ref, a0_ref, a1_ref, z_ref, ws_ref, wm_ref, ze_ref, out_ref = refs
    else:
      (ha_ref, a0_ref, a1_ref, z_ref, ws_ref, wm_ref, ze_ref, wo_ref,
       out_ref) = refs
    h = ha_ref[...]
    agg = (a0_ref[...] + a1_ref[...]) * np.float32(1.0 / np.sqrt(NNEI))
    z = z_ref[...]
    spec = lax.broadcasted_iota(jnp.int32, (BN, NSPEC), 1)
    oneh = (z == spec).astype(jnp.float32)
    acc = (jnp.dot(h, ws_ref[...], preferred_element_type=jnp.float32)
           + jnp.dot(agg, wm_ref[...], preferred_element_type=jnp.float32)
           + jnp.dot(oneh, ze_ref[...], preferred_element_type=jnp.float32))
    hn = acc * (1.0 / (1.0 + jnp.exp(-acc)))
    if mode == "mid":
      out_ref[...] = hn
      return
    out = jnp.dot(hn, wo_ref[...], preferred_element_type=jnp.float32)
    if mode == "out_avg":
      out = (out + prev_ref[...]) * 0.5
    elif mode == "out_abs":
      out = jnp.abs(out)
    out_ref[...] = out

  nd = pl.BlockSpec((BN, D), lambda i: (i, 0))
  dd = pl.BlockSpec((D, D), lambda i: (0, 0))
  in_specs = [nd, nd, nd,
              pl.BlockSpec((BN, 1), lambda i: (i, 0)),
              dd, dd,
              pl.BlockSpec((NSPEC, D), lambda i: (0, 0))]
  if mode != "mid":
    in_specs.append(dd)
  if mode == "out_avg":
    in_specs.append(nd)
  return pl.pallas_call(
      body,
      grid=(N // BN,),
      in_specs=in_specs,
      out_specs=nd,
      out_shape=jax.ShapeDtypeStruct((N, D), jnp.float32),
  )


_gather_pos2 = _make_sc_gather2()
_fused_k = [_make_sc_fused(k) for k in range(6)]
_wall_k = _make_tc_wall()
_node_mid_k = _make_tc_node("mid")
_node_out_k = _make_tc_node("out")
_node_out_avg_k = _make_tc_node("out_avg")
_node_out_abs_k = _make_tc_node("out_abs")


def kernel(pos, x, pos_final_state, x_final_state,
           pos_interpolated_transition_state, species_initial_state,
           species_final_state, batch, edge_index, Wself, Wmsg, R1, R2, Wout,
           Zemb):
  postab = jnp.concatenate(
      [pos, pos_final_state, pos_interpolated_transition_state,
       jnp.zeros((N, D - 9), jnp.float32)], axis=1)
  src_i = edge_index[0].astype(jnp.int32)
  dst_i = edge_index[1].astype(jnp.int32)
  src3 = src_i.reshape(NW, NCHUNK, CH)
  dst3 = dst_i.reshape(NW, NCHUNK, CH)
  src4 = src_i.reshape(NW, NWIN, WWIN, CH2)
  dst4 = dst_i.reshape(NW, NWIN, WWIN, CH2)
  z_init = species_initial_state.astype(jnp.int32).reshape(N, 1)
  z_final = species_final_state.astype(jnp.int32).reshape(N, 1)
  zeros_nd = jnp.zeros((N, D), jnp.float32)
  zeros_pad = jnp.zeros((NPAD, D), jnp.float32)

  possrc, posdst = _gather_pos2(postab, src3, dst3)
  r1cat = jnp.concatenate([R1[:, 0], R1[:, 1]], axis=2).astype(jnp.bfloat16)
  zblk = jnp.zeros((3, RN, D), jnp.float32)
  r2bd = jnp.concatenate(
      [jnp.concatenate([R2[:, 0], zblk], axis=2),
       jnp.concatenate([zblk, R2[:, 1]], axis=2)],
      axis=1).astype(jnp.bfloat16)
  wall = _wall_k(possrc, posdst, r1cat, r2bd).reshape(6 * E, D)

  def seg(which, h):
    return _fused_k[which](wall, h, src4, dst4, zeros_pad)[:, :N]

  a00 = seg(0, x)
  a10 = seg(2, x_final_state)
  h0 = _node_mid_k(x, a00[0], a00[1], z_init, Wself[0, 0], Wmsg[0, 0],
                   Zemb[0])
  a01 = seg(1, h0)
  h1 = _node_mid_k(x_final_state, a10[0], a10[1], z_final, Wself[1, 0],
                   Wmsg[1, 0], Zemb[1])
  a11 = seg(3, h1)
  out_init = _node_out_k(h0, a01[0], a01[1], z_init, Wself[0, 1], Wmsg[0, 1],
                         Zemb[0], Wout[0])
  x_ts = _node_out_avg_k(h1, a11[0], a11[1], z_final, Wself[1, 1],
                         Wmsg[1, 1], Zemb[1], Wout[1], out_init)
  a20 = seg(4, x_ts)
  h2 = _node_mid_k(x_ts, a20[0], a20[1], z_init, Wself[2, 0], Wmsg[2, 0],
                   Zemb[2])
  a21 = seg(5, h2)
  return _node_out_abs_k(h2, a21[0], a21[1], z_init, Wself[2, 1],
                         Wmsg[2, 1], Zemb[2], Wout[2])


# speedup vs baseline: 5.0073x; 1.0052x over previous
import functools

import jax
import jax.numpy as jnp
import numpy as np
from jax import lax
from jax.experimental import pallas as pl
from jax.experimental.pallas import tpu as pltpu
from jax.experimental.pallas import tpu_sc as plsc

N = 10000
E = 320000
D = 128
NB = 16
RN = 64
MAXR = 5.0
NNEI = 32.0
NSPEC = 10

NC = 2
NS = 16
NW = NC * NS
EW = E // NW
CH = 80
NCHUNK = EW // CH
NPAD = 10240
ROWS_PER_TILE = NPAD // NS

_mesh = plsc.VectorSubcoreMesh(
    core_axis_name="c", subcore_axis_name="s", num_cores=NC, num_subcores=NS)


def _worker_id():
  return lax.axis_index("c") * NS + lax.axis_index("s")


def _make_sc_gather2():
  @functools.partial(
      pl.kernel,
      out_type=(jax.ShapeDtypeStruct((E, D), jnp.float32),
                jax.ShapeDtypeStruct((E, D), jnp.float32)),
      mesh=_mesh,
      scratch_types=[
          pltpu.VMEM((NCHUNK, CH), jnp.int32),
          pltpu.VMEM((NCHUNK, CH), jnp.int32),
          pltpu.VMEM((CH, D), jnp.float32),
          pltpu.VMEM((CH, D), jnp.float32),
          pltpu.VMEM((CH, D), jnp.float32),
          pltpu.VMEM((CH, D), jnp.float32),
          pltpu.SemaphoreType.DMA,
          pltpu.SemaphoreType.DMA,
      ],
  )
  def gather_kernel(table_hbm, sidx_hbm, didx_hbm, outs_hbm, outd_hbm,
                    sidx_v, didx_v, sbuf0, sbuf1, dbuf0, dbuf1, sem0, sem1):
    wid = _worker_id()
    pltpu.sync_copy(sidx_hbm.at[wid], sidx_v)
    pltpu.sync_copy(didx_hbm.at[wid], didx_v)
    ebase = wid * EW
    sbufs = (sbuf0, sbuf1)
    dbufs = (dbuf0, dbuf1)
    sems = (sem0, sem1)

    def issue(j, p):
      pltpu.async_copy(table_hbm.at[sidx_v.at[j]], sbufs[p], sems[p])
      pltpu.async_copy(table_hbm.at[didx_v.at[j]], dbufs[p], sems[p])

    def drain(j, p):
      pltpu.make_async_copy(table_hbm.at[sidx_v.at[j]], sbufs[p],
                            sems[p]).wait()
      pltpu.make_async_copy(table_hbm.at[didx_v.at[j]], dbufs[p],
                            sems[p]).wait()
      pltpu.sync_copy(sbufs[p], outs_hbm.at[pl.ds(ebase + j * CH, CH)])
      pltpu.sync_copy(dbufs[p], outd_hbm.at[pl.ds(ebase + j * CH, CH)])

    issue(0, 0)

    def body(j, _):
      for p in range(2):
        jj = 2 * j + p
        issue(jj + 1, 1 - p)
        drain(jj, p)
      return 0

    lax.fori_loop(0, (NCHUNK - 1) // 2, body, 0)
    drain(NCHUNK - 1, 0)

  return gather_kernel


CH2 = 40
NCH2 = EW // CH2
WWIN = 50
NWIN = NCH2 // WWIN


def _make_sc_fused(which):
  wbase0 = which * E

  @functools.partial(
      pl.kernel,
      out_type=jax.ShapeDtypeStruct((NC, NPAD, D), jnp.float32),
      mesh=_mesh,
      scratch_types=[
          pltpu.VMEM((WWIN, CH2), jnp.int32),
          pltpu.VMEM((WWIN, CH2), jnp.int32),
          pltpu.VMEM((CH2, D), jnp.float32),
          pltpu.VMEM((CH2, D), jnp.float32),
          pltpu.VMEM((CH2, D), jnp.float32),
          pltpu.VMEM((CH2, D), jnp.float32),
          pltpu.VMEM((CH2, D), jnp.float32),
          pltpu.VMEM((CH2, D), jnp.float32),
          pltpu.VMEM_SHARED((NPAD, D), jnp.float32),
          pltpu.SemaphoreType.DMA,
          pltpu.SemaphoreType.DMA,
          pltpu.SemaphoreType.DMA,
          pltpu.SemaphoreType.DMA,
      ],
  )
  def fused_kernel(w6_hbm, h_hbm, src_hbm, dst_hbm, zeros_hbm, out_hbm,
                   src_v, dst_v, wb0, wb1, hb0, hb1, mb0, mb1, acc_sh,
                   sem0, sem1, scs0, scs1):
    c = lax.axis_index("c")
    s = lax.axis_index("s")
    wid = c * NS + s
    rbase = s * ROWS_PER_TILE
    pltpu.sync_copy(zeros_hbm.at[pl.ds(rbase, ROWS_PER_TILE)],
                    acc_sh.at[pl.ds(rbase, ROWS_PER_TILE)])
    plsc.subcore_barrier()
    ebase = wid * EW
    wbase = wbase0 + ebase
    wbufs = (wb0, wb1)
    hbufs = (hb0, hb1)
    mbufs = (mb0, mb1)
    sems = (sem0, sem1)
    scsems = (scs0, scs1)

    def win_body(win, _):
      pltpu.sync_copy(src_hbm.at[wid, win], src_v)
      pltpu.sync_copy(dst_hbm.at[wid, win], dst_v)
      wb_e = wbase + win * (WWIN * CH2)

      def issue(j, p):
        pltpu.async_copy(w6_hbm.at[pl.ds(wb_e + j * CH2, CH2)], wbufs[p],
                         sems[p])
        pltpu.async_copy(h_hbm.at[src_v.at[j]], hbufs[p], sems[p])

      def wait(j, p):
        pltpu.make_async_copy(w6_hbm.at[pl.ds(wb_e + j * CH2, CH2)],
                              wbufs[p], sems[p]).wait()
        pltpu.make_async_copy(h_hbm.at[src_v.at[j]], hbufs[p],
                              sems[p]).wait()

      def wait_sc(j, p):
        pltpu.make_async_copy(mbufs[p], acc_sh.at[dst_v.at[j]],
                              scsems[p]).wait()

      def process(j, p):
        wb = wbufs[p]
        hb = hbufs[p]
        mb = mbufs[p]

        def mrow(i, _):
          for k in range(D // 16):
            sl = pl.ds(k * 16, 16)
            mb[i, sl] = wb[i, sl] * hb[i, sl]
          return 0

        lax.fori_loop(0, CH2, mrow, 0)
        pltpu.async_copy(mb, acc_sh.at[dst_v.at[j]], scsems[p], add=True)

      issue(0, 0)

      def pair(i, _):
        j0 = 2 * i
        issue(j0 + 1, 1)
        wait(j0, 0)

        @pl.when(i > 0)
        def _():
          wait_sc(j0 - 2, 0)

        process(j0, 0)

        @pl.when(j0 + 2 < WWIN)
        def _():
          issue(j0 + 2, 0)

        wait(j0 + 1, 1)

        @pl.when(i > 0)
        def _():
          wait_sc(j0 - 1, 1)

        process(j0 + 1, 1)
        return 0

      lax.fori_loop(0, WWIN // 2, pair, 0)
      wait_sc(WWIN - 2, 0)
      wait_sc(WWIN - 1, 1)
      return 0

    lax.fori_loop(0, NWIN, win_body, 0)
    plsc.subcore_barrier()
    pltpu.sync_copy(acc_sh.at[pl.ds(rbase, ROWS_PER_TILE)],
                    out_hbm.at[c, pl.ds(rbase, ROWS_PER_TILE)])

  return fused_kernel


BE = 2000
_SIG = MAXR / NB
_INV2SIG2 = np.float32(1.0 / (2.0 * _SIG * _SIG))
_CSTEP = np.float32(MAXR / (NB - 1))


def _make_tc_wall():
  def body(ps_ref, pd_ref, r1_ref, r2_ref, out_ref):
    centers = lax.broadcasted_iota(jnp.int32, (1, NB), 1).astype(
        jnp.float32) * _CSTEP
    diff = ps_ref[:, :NB] - pd_ref[:, :NB]
    dsq = diff * diff
    colsel = lax.broadcasted_iota(jnp.int32, (NB, 3), 0)
    netsel = lax.broadcasted_iota(jnp.int32, (NB, 3), 1)
    sel = ((colsel >= 3 * netsel) & (colsel < 3 * netsel + 3)).astype(
        jnp.float32)
    d2all = jnp.dot(dsq, sel, preferred_element_type=jnp.float32) + 1e-12
    rall = jnp.sqrt(d2all)
    t = jnp.minimum(rall * (1.0 / MAXR), 1.0)
    u = t * t
    cosp = np.float32(0.0016053627762021867)
    for cc in (-0.025391111383297586, 0.2350633717621909,
               -1.3351744534102399, 4.0586982622690035,
               -4.934801388370911, 0.9999999922898454):
      cosp = cosp * u + np.float32(cc)
    cutall = 0.5 * (cosp + 1.0)
    for net in range(3):
      r = rall[:, net:net + 1]
      cut = cutall[:, net:net + 1]
      rb = jnp.exp2(-((r - centers) ** 2)
                    * np.float32(_INV2SIG2 * 1.4426950408889634))
      rb = (rb * cut).astype(jnp.bfloat16)
      u = jnp.dot(rb, r1_ref[net], preferred_element_type=jnp.float32)
      u = (u / (1.0 + jnp.exp2(u * np.float32(-1.4426950408889634)))
           ).astype(jnp.bfloat16)
      w2 = jnp.dot(u, r2_ref[net], preferred_element_type=jnp.float32)
      out_ref[2 * net] = w2[:, :D]
      out_ref[2 * net + 1] = w2[:, D:]

  return pl.pallas_call(
      body,
      grid=(E // BE,),
      in_specs=[
          pl.BlockSpec((BE, D), lambda i: (i, 0)),
          pl.BlockSpec((BE, D), lambda i: (i, 0)),
          pl.BlockSpec((3, NB, 2 * RN), lambda i: (0, 0, 0)),
          pl.BlockSpec((3, 2 * RN, 2 * D), lambda i: (0, 0, 0)),
      ],
      out_specs=pl.BlockSpec((6, BE, D), lambda i: (0, i, 0)),
      out_shape=jax.ShapeDtypeStruct((6, E, D), jnp.float32),
  )


BN = 1000


def _make_tc_node(mode):
  def body(*refs):
    if mode == "out_avg":
      (ha_ref, a0_ref, a1_ref, z_ref, ws_ref, wm_ref, ze_ref, wo_ref,
       prev_ref, out_ref) = refs
    elif mode == "mid":
      ha_ref, a0_ref, a1_ref, z_ref, ws_ref, wm_ref, ze_ref, out_ref = refs
    else:
      (ha_ref, a0_ref, a1_ref, z_ref, ws_ref, wm_ref, ze_ref, wo_ref,
       out_ref) = refs
    h = ha_ref[...]
    agg = (a0_ref[...] + a1_ref[...]) * np.float32(1.0 / np.sqrt(NNEI))
    z = z_ref[...]
    spec = lax.broadcasted_iota(jnp.int32, (BN, NSPEC), 1)
    oneh = (z == spec).astype(jnp.float32)
    acc = (jnp.dot(h, ws_ref[...], preferred_element_type=jnp.float32)
           + jnp.dot(agg, wm_ref[...], preferred_element_type=jnp.float32)
           + jnp.dot(oneh, ze_ref[...], preferred_element_type=jnp.float32))
    hn = acc * (1.0 / (1.0 + jnp.exp(-acc)))
    if mode == "mid":
      out_ref[...] = hn
      return
    out = jnp.dot(hn, wo_ref[...], preferred_element_type=jnp.float32)
    if mode == "out_avg":
      out = (out + prev_ref[...]) * 0.5
    elif mode == "out_abs":
      out = jnp.abs(out)
    out_ref[...] = out

  nd = pl.BlockSpec((BN, D), lambda i: (i, 0))
  dd = pl.BlockSpec((D, D), lambda i: (0, 0))
  in_specs = [nd, nd, nd,
              pl.BlockSpec((BN, 1), lambda i: (i, 0)),
              dd, dd,
              pl.BlockSpec((NSPEC, D), lambda i: (0, 0))]
  if mode != "mid":
    in_specs.append(dd)
  if mode == "out_avg":
    in_specs.append(nd)
  return pl.pallas_call(
      body,
      grid=(N // BN,),
      in_specs=in_specs,
      out_specs=nd,
      out_shape=jax.ShapeDtypeStruct((N, D), jnp.float32),
  )


_gather_pos2 = _make_sc_gather2()
_fused_k = [_make_sc_fused(k) for k in range(6)]
_wall_k = _make_tc_wall()
_node_mid_k = _make_tc_node("mid")
_node_out_k = _make_tc_node("out")
_node_out_avg_k = _make_tc_node("out_avg")
_node_out_abs_k = _make_tc_node("out_abs")


def kernel(pos, x, pos_final_state, x_final_state,
           pos_interpolated_transition_state, species_initial_state,
           species_final_state, batch, edge_index, Wself, Wmsg, R1, R2, Wout,
           Zemb):
  postab = jnp.concatenate(
      [pos, pos_final_state, pos_interpolated_transition_state,
       jnp.zeros((N, D - 9), jnp.float32)], axis=1)
  src_i = edge_index[0].astype(jnp.int32)
  dst_i = edge_index[1].astype(jnp.int32)
  src3 = src_i.reshape(NW, NCHUNK, CH)
  dst3 = dst_i.reshape(NW, NCHUNK, CH)
  src4 = src_i.reshape(NW, NWIN, WWIN, CH2)
  dst4 = dst_i.reshape(NW, NWIN, WWIN, CH2)
  z_init = species_initial_state.astype(jnp.int32).reshape(N, 1)
  z_final = species_final_state.astype(jnp.int32).reshape(N, 1)
  zeros_nd = jnp.zeros((N, D), jnp.float32)
  zeros_pad = jnp.zeros((NPAD, D), jnp.float32)

  possrc, posdst = _gather_pos2(postab, src3, dst3)
  r1cat = jnp.concatenate([R1[:, 0], R1[:, 1]], axis=2).astype(jnp.bfloat16)
  zblk = jnp.zeros((3, RN, D), jnp.float32)
  r2bd = jnp.concatenate(
      [jnp.concatenate([R2[:, 0], zblk], axis=2),
       jnp.concatenate([zblk, R2[:, 1]], axis=2)],
      axis=1).astype(jnp.bfloat16)
  wall = _wall_k(possrc, posdst, r1cat, r2bd).reshape(6 * E, D)

  def seg(which, h):
    return _fused_k[which](wall, h, src4, dst4, zeros_pad)[:, :N]

  a00 = seg(0, x)
  a10 = seg(2, x_final_state)
  h0 = _node_mid_k(x, a00[0], a00[1], z_init, Wself[0, 0], Wmsg[0, 0],
                   Zemb[0])
  a01 = seg(1, h0)
  h1 = _node_mid_k(x_final_state, a10[0], a10[1], z_final, Wself[1, 0],
                   Wmsg[1, 0], Zemb[1])
  a11 = seg(3, h1)
  out_init = _node_out_k(h0, a01[0], a01[1], z_init, Wself[0, 1], Wmsg[0, 1],
                         Zemb[0], Wout[0])
  x_ts = _node_out_avg_k(h1, a11[0], a11[1], z_final, Wself[1, 1],
                         Wmsg[1, 1], Zemb[1], Wout[1], out_init)
  a20 = seg(4, x_ts)
  h2 = _node_mid_k(x_ts, a20[0], a20[1], z_init, Wself[2, 0], Wmsg[2, 0],
                   Zemb[2])
  a21 = seg(5, h2)
  return _node_out_abs_k(h2, a21[0], a21[1], z_init, Wself[2, 1],
                         Wmsg[2, 1], Zemb[2], Wout[2])


# speedup vs baseline: 5.0276x; 1.0041x over previous
import functools

import jax
import jax.numpy as jnp
import numpy as np
from jax import lax
from jax.experimental import pallas as pl
from jax.experimental.pallas import tpu as pltpu
from jax.experimental.pallas import tpu_sc as plsc

N = 10000
E = 320000
D = 128
NB = 16
RN = 64
MAXR = 5.0
NNEI = 32.0
NSPEC = 10

NC = 2
NS = 16
NW = NC * NS
EW = E // NW
CH = 80
NCHUNK = EW // CH
NPAD = 10240
ROWS_PER_TILE = NPAD // NS

_mesh = plsc.VectorSubcoreMesh(
    core_axis_name="c", subcore_axis_name="s", num_cores=NC, num_subcores=NS)


def _worker_id():
  return lax.axis_index("c") * NS + lax.axis_index("s")


def _make_sc_gather2():
  @functools.partial(
      pl.kernel,
      out_type=(jax.ShapeDtypeStruct((E, D), jnp.float32),
                jax.ShapeDtypeStruct((E, D), jnp.float32)),
      mesh=_mesh,
      scratch_types=[
          pltpu.VMEM((NCHUNK, CH), jnp.int32),
          pltpu.VMEM((NCHUNK, CH), jnp.int32),
          pltpu.VMEM((CH, D), jnp.float32),
          pltpu.VMEM((CH, D), jnp.float32),
          pltpu.VMEM((CH, D), jnp.float32),
          pltpu.VMEM((CH, D), jnp.float32),
          pltpu.SemaphoreType.DMA,
          pltpu.SemaphoreType.DMA,
      ],
  )
  def gather_kernel(table_hbm, sidx_hbm, didx_hbm, outs_hbm, outd_hbm,
                    sidx_v, didx_v, sbuf0, sbuf1, dbuf0, dbuf1, sem0, sem1):
    wid = _worker_id()
    pltpu.sync_copy(sidx_hbm.at[wid], sidx_v)
    pltpu.sync_copy(didx_hbm.at[wid], didx_v)
    ebase = wid * EW
    sbufs = (sbuf0, sbuf1)
    dbufs = (dbuf0, dbuf1)
    sems = (sem0, sem1)

    def issue(j, p):
      pltpu.async_copy(table_hbm.at[sidx_v.at[j]], sbufs[p], sems[p])
      pltpu.async_copy(table_hbm.at[didx_v.at[j]], dbufs[p], sems[p])

    def drain(j, p):
      pltpu.make_async_copy(table_hbm.at[sidx_v.at[j]], sbufs[p],
                            sems[p]).wait()
      pltpu.make_async_copy(table_hbm.at[didx_v.at[j]], dbufs[p],
                            sems[p]).wait()
      pltpu.sync_copy(sbufs[p], outs_hbm.at[pl.ds(ebase + j * CH, CH)])
      pltpu.sync_copy(dbufs[p], outd_hbm.at[pl.ds(ebase + j * CH, CH)])

    issue(0, 0)

    def body(j, _):
      for p in range(2):
        jj = 2 * j + p
        issue(jj + 1, 1 - p)
        drain(jj, p)
      return 0

    lax.fori_loop(0, (NCHUNK - 1) // 2, body, 0)
    drain(NCHUNK - 1, 0)

  return gather_kernel


CH2 = 40
NCH2 = EW // CH2
WWIN = 50
NWIN = NCH2 // WWIN


def _make_sc_fused(which):
  wbase0 = which * E

  @functools.partial(
      pl.kernel,
      out_type=jax.ShapeDtypeStruct((NC, NPAD, D), jnp.float32),
      mesh=_mesh,
      scratch_types=[
          pltpu.VMEM((WWIN, CH2), jnp.int32),
          pltpu.VMEM((WWIN, CH2), jnp.int32),
          pltpu.VMEM((CH2, D), jnp.float32),
          pltpu.VMEM((CH2, D), jnp.float32),
          pltpu.VMEM((CH2, D), jnp.float32),
          pltpu.VMEM((CH2, D), jnp.float32),
          pltpu.VMEM((CH2, D), jnp.float32),
          pltpu.VMEM((CH2, D), jnp.float32),
          pltpu.VMEM_SHARED((NPAD, D), jnp.float32),
          pltpu.SemaphoreType.DMA,
          pltpu.SemaphoreType.DMA,
          pltpu.SemaphoreType.DMA,
          pltpu.SemaphoreType.DMA,
      ],
  )
  def fused_kernel(w6_hbm, h_hbm, src_hbm, dst_hbm, zeros_hbm, out_hbm,
                   src_v, dst_v, wb0, wb1, hb0, hb1, mb0, mb1, acc_sh,
                   sem0, sem1, scs0, scs1):
    c = lax.axis_index("c")
    s = lax.axis_index("s")
    wid = c * NS + s
    rbase = s * ROWS_PER_TILE
    pltpu.sync_copy(zeros_hbm.at[pl.ds(rbase, ROWS_PER_TILE)],
                    acc_sh.at[pl.ds(rbase, ROWS_PER_TILE)])
    plsc.subcore_barrier()
    ebase = wid * EW
    wbase = wbase0 + ebase
    wbufs = (wb0, wb1)
    hbufs = (hb0, hb1)
    mbufs = (mb0, mb1)
    sems = (sem0, sem1)
    scsems = (scs0, scs1)

    def win_body(win, _):
      pltpu.sync_copy(src_hbm.at[wid, win], src_v)
      pltpu.sync_copy(dst_hbm.at[wid, win], dst_v)
      wb_e = wbase + win * (WWIN * CH2)

      def issue(j, p):
        pltpu.async_copy(w6_hbm.at[pl.ds(wb_e + j * CH2, CH2)], wbufs[p],
                         sems[p])
        pltpu.async_copy(h_hbm.at[src_v.at[j]], hbufs[p], sems[p])

      def wait(j, p):
        pltpu.make_async_copy(w6_hbm.at[pl.ds(wb_e + j * CH2, CH2)],
                              wbufs[p], sems[p]).wait()
        pltpu.make_async_copy(h_hbm.at[src_v.at[j]], hbufs[p],
                              sems[p]).wait()

      def wait_sc(j, p):
        pltpu.make_async_copy(mbufs[p], acc_sh.at[dst_v.at[j]],
                              scsems[p]).wait()

      def process(j, p):
        wb = wbufs[p]
        hb = hbufs[p]
        mb = mbufs[p]

        def mrow(i, _):
          for r in range(2):
            for k in range(D // 16):
              sl = pl.ds(k * 16, 16)
              mb[2 * i + r, sl] = wb[2 * i + r, sl] * hb[2 * i + r, sl]
          return 0

        lax.fori_loop(0, CH2 // 2, mrow, 0)
        pltpu.async_copy(mb, acc_sh.at[dst_v.at[j]], scsems[p], add=True)

      issue(0, 0)

      def pair(i, _):
        j0 = 2 * i
        issue(j0 + 1, 1)
        wait(j0, 0)

        @pl.when(i > 0)
        def _():
          wait_sc(j0 - 2, 0)

        process(j0, 0)

        @pl.when(j0 + 2 < WWIN)
        def _():
          issue(j0 + 2, 0)

        wait(j0 + 1, 1)

        @pl.when(i > 0)
        def _():
          wait_sc(j0 - 1, 1)

        process(j0 + 1, 1)
        return 0

      lax.fori_loop(0, WWIN // 2, pair, 0)
      wait_sc(WWIN - 2, 0)
      wait_sc(WWIN - 1, 1)
      return 0

    lax.fori_loop(0, NWIN, win_body, 0)
    plsc.subcore_barrier()
    pltpu.sync_copy(acc_sh.at[pl.ds(rbase, ROWS_PER_TILE)],
                    out_hbm.at[c, pl.ds(rbase, ROWS_PER_TILE)])

  return fused_kernel


BE = 2000
_SIG = MAXR / NB
_INV2SIG2 = np.float32(1.0 / (2.0 * _SIG * _SIG))
_CSTEP = np.float32(MAXR / (NB - 1))


def _make_tc_wall():
  def body(ps_ref, pd_ref, r1_ref, r2_ref, out_ref):
    centers = lax.broadcasted_iota(jnp.int32, (1, NB), 1).astype(
        jnp.float32) * _CSTEP
    diff = ps_ref[:, :NB] - pd_ref[:, :NB]
    dsq = diff * diff
    colsel = lax.broadcasted_iota(jnp.int32, (NB, 3), 0)
    netsel = lax.broadcasted_iota(jnp.int32, (NB, 3), 1)
    sel = ((colsel >= 3 * netsel) & (colsel < 3 * netsel + 3)).astype(
        jnp.float32)
    d2all = jnp.dot(dsq, sel, preferred_element_type=jnp.float32) + 1e-12
    rall = jnp.sqrt(d2all)
    t = jnp.minimum(rall * (1.0 / MAXR), 1.0)
    u = t * t
    cosp = np.float32(0.0016053627762021867)
    for cc in (-0.025391111383297586, 0.2350633717621909,
               -1.3351744534102399, 4.0586982622690035,
               -4.934801388370911, 0.9999999922898454):
      cosp = cosp * u + np.float32(cc)
    cutall = 0.5 * (cosp + 1.0)
    for net in range(3):
      r = rall[:, net:net + 1]
      cut = cutall[:, net:net + 1]
      rb = jnp.exp2(-((r - centers) ** 2)
                    * np.float32(_INV2SIG2 * 1.4426950408889634))
      rb = (rb * cut).astype(jnp.bfloat16)
      u = jnp.dot(rb, r1_ref[net], preferred_element_type=jnp.float32)
      u = (u / (1.0 + jnp.exp2(u * np.float32(-1.4426950408889634)))
           ).astype(jnp.bfloat16)
      w2 = jnp.dot(u, r2_ref[net], preferred_element_type=jnp.float32)
      out_ref[2 * net] = w2[:, :D]
      out_ref[2 * net + 1] = w2[:, D:]

  return pl.pallas_call(
      body,
      grid=(E // BE,),
      in_specs=[
          pl.BlockSpec((BE, D), lambda i: (i, 0)),
          pl.BlockSpec((BE, D), lambda i: (i, 0)),
          pl.BlockSpec((3, NB, 2 * RN), lambda i: (0, 0, 0)),
          pl.BlockSpec((3, 2 * RN, 2 * D), lambda i: (0, 0, 0)),
      ],
      out_specs=pl.BlockSpec((6, BE, D), lambda i: (0, i, 0)),
      out_shape=jax.ShapeDtypeStruct((6, E, D), jnp.float32),
  )


BN = 2000


def _make_tc_node(mode):
  def body(*refs):
    if mode == "out_avg":
      (ha_ref, a0_ref, a1_ref, z_ref, ws_ref, wm_ref, ze_ref, wo_ref,
       prev_ref, out_ref) = refs
    elif mode == "mid":
      ha_ref, a0_ref, a1_ref, z_ref, ws_ref, wm_ref, ze_ref, out_ref = refs
    else:
      (ha_ref, a0_ref, a1_ref, z_ref, ws_ref, wm_ref, ze_ref, wo_ref,
       out_ref) = refs
    h = ha_ref[...]
    agg = (a0_ref[...] + a1_ref[...]) * np.float32(1.0 / np.sqrt(NNEI))
    z = z_ref[...]
    spec = lax.broadcasted_iota(jnp.int32, (BN, NSPEC), 1)
    oneh = (z == spec).astype(jnp.float32)
    acc = (jnp.dot(h, ws_ref[...], preferred_element_type=jnp.float32)
           + jnp.dot(agg, wm_ref[...], preferred_element_type=jnp.float32)
           + jnp.dot(oneh, ze_ref[...], preferred_element_type=jnp.float32))
    hn = acc * (1.0 / (1.0 + jnp.exp(-acc)))
    if mode == "mid":
      out_ref[...] = hn
      return
    out = jnp.dot(hn, wo_ref[...], preferred_element_type=jnp.float32)
    if mode == "out_avg":
      out = (out + prev_ref[...]) * 0.5
    elif mode == "out_abs":
      out = jnp.abs(out)
    out_ref[...] = out

  nd = pl.BlockSpec((BN, D), lambda i: (i, 0))
  dd = pl.BlockSpec((D, D), lambda i: (0, 0))
  in_specs = [nd, nd, nd,
              pl.BlockSpec((BN, 1), lambda i: (i, 0)),
              dd, dd,
              pl.BlockSpec((NSPEC, D), lambda i: (0, 0))]
  if mode != "mid":
    in_specs.append(dd)
  if mode == "out_avg":
    in_specs.append(nd)
  return pl.pallas_call(
      body,
      grid=(N // BN,),
      in_specs=in_specs,
      out_specs=nd,
      out_shape=jax.ShapeDtypeStruct((N, D), jnp.float32),
  )


_gather_pos2 = _make_sc_gather2()
_fused_k = [_make_sc_fused(k) for k in range(6)]
_wall_k = _make_tc_wall()
_node_mid_k = _make_tc_node("mid")
_node_out_k = _make_tc_node("out")
_node_out_avg_k = _make_tc_node("out_avg")
_node_out_abs_k = _make_tc_node("out_abs")


def kernel(pos, x, pos_final_state, x_final_state,
           pos_interpolated_transition_state, species_initial_state,
           species_final_state, batch, edge_index, Wself, Wmsg, R1, R2, Wout,
           Zemb):
  postab = jnp.concatenate(
      [pos, pos_final_state, pos_interpolated_transition_state,
       jnp.zeros((N, D - 9), jnp.float32)], axis=1)
  src_i = edge_index[0].astype(jnp.int32)
  dst_i = edge_index[1].astype(jnp.int32)
  src3 = src_i.reshape(NW, NCHUNK, CH)
  dst3 = dst_i.reshape(NW, NCHUNK, CH)
  src4 = src_i.reshape(NW, NWIN, WWIN, CH2)
  dst4 = dst_i.reshape(NW, NWIN, WWIN, CH2)
  z_init = species_initial_state.astype(jnp.int32).reshape(N, 1)
  z_final = species_final_state.astype(jnp.int32).reshape(N, 1)
  zeros_nd = jnp.zeros((N, D), jnp.float32)
  zeros_pad = jnp.zeros((NPAD, D), jnp.float32)

  possrc, posdst = _gather_pos2(postab, src3, dst3)
  r1cat = jnp.concatenate([R1[:, 0], R1[:, 1]], axis=2).astype(jnp.bfloat16)
  zblk = jnp.zeros((3, RN, D), jnp.float32)
  r2bd = jnp.concatenate(
      [jnp.concatenate([R2[:, 0], zblk], axis=2),
       jnp.concatenate([zblk, R2[:, 1]], axis=2)],
      axis=1).astype(jnp.bfloat16)
  wall = _wall_k(possrc, posdst, r1cat, r2bd).reshape(6 * E, D)

  def seg(which, h):
    return _fused_k[which](wall, h, src4, dst4, zeros_pad)[:, :N]

  a00 = seg(0, x)
  a10 = seg(2, x_final_state)
  h0 = _node_mid_k(x, a00[0], a00[1], z_init, Wself[0, 0], Wmsg[0, 0],
                   Zemb[0])
  a01 = seg(1, h0)
  h1 = _node_mid_k(x_final_state, a10[0], a10[1], z_final, Wself[1, 0],
                   Wmsg[1, 0], Zemb[1])
  a11 = seg(3, h1)
  out_init = _node_out_k(h0, a01[0], a01[1], z_init, Wself[0, 1], Wmsg[0, 1],
                         Zemb[0], Wout[0])
  x_ts = _node_out_avg_k(h1, a11[0], a11[1], z_final, Wself[1, 1],
                         Wmsg[1, 1], Zemb[1], Wout[1], out_init)
  a20 = seg(4, x_ts)
  h2 = _node_mid_k(x_ts, a20[0], a20[1], z_init, Wself[2, 0], Wmsg[2, 0],
                   Zemb[2])
  a21 = seg(5, h2)
  return _node_out_abs_k(h2, a21[0], a21[1], z_init, Wself[2, 1],
                         Wmsg[2, 1], Zemb[2], Wout[2])
